# Initial kernel scaffold; baseline (speedup 1.0000x reference)
import jax
import jax.numpy as jnp
from jax.experimental import pallas as pl

N = 100000
E = 1600000
D_IN = 128
HID = 16
HEADS = 2
OUT = 16
EDGE_DIM = 8

NB = 2500
N_GRID = N // NB
EB = 12500
E_GRID = E // EB


def _k1_body(x_ref, w1_ref, b1_ref, t1_ref, s_ref):
    i = pl.program_id(0)
    t1 = jnp.dot(x_ref[...], w1_ref[...], preferred_element_type=jnp.float32)
    t1 = t1 + b1_ref[...][None, :]
    t1_ref[...] = t1

    @pl.when(i == 0)
    def _():
        s_ref[...] = jnp.zeros_like(s_ref)

    s_ref[0:1, :] += jnp.sum(t1, axis=0)[None, :]
    s_ref[1:2, :] += jnp.sum(t1 * t1, axis=0)[None, :]


def _k2_body(t1_ref, s1_ref, w2_ref, b2_ref, g1_ref, be1_ref, t2_ref, s_ref):
    i = pl.program_id(0)
    s1 = s1_ref[...]
    m = s1[0, :] / N
    v = s1[1, :] / N - m * m
    rstd = jax.lax.rsqrt(v + 1e-5)
    h1 = (t1_ref[...] - m[None, :]) * (rstd * g1_ref[...])[None, :] + be1_ref[...][None, :]
    h1 = jnp.maximum(h1, 0.0)
    t2 = jnp.dot(h1, w2_ref[...], preferred_element_type=jnp.float32)
    t2 = t2 + b2_ref[...][None, :]
    t2_ref[...] = t2

    @pl.when(i == 0)
    def _():
        s_ref[...] = jnp.zeros_like(s_ref)

    s_ref[0:1, :] += jnp.sum(t2, axis=0)[None, :]
    s_ref[1:2, :] += jnp.sum(t2 * t2, axis=0)[None, :]


def _k3_body(t2_ref, s2_ref, g_ref, be_ref, wc_ref, att_ref,
             xw_ref, asd_ref, mx_ref):
    i = pl.program_id(0)
    s2 = s2_ref[...]
    m = s2[0, :] / N
    v = s2[1, :] / N - m * m
    rstd = jax.lax.rsqrt(v + 1e-5)
    h = (t2_ref[...] - m[None, :]) * (rstd * g_ref[...])[None, :] + be_ref[...][None, :]
    h = jnp.maximum(h, 0.0)
    xw = jnp.dot(h, wc_ref[...], preferred_element_type=jnp.float32)
    xw_ref[...] = xw
    asd = jnp.dot(xw, att_ref[...], preferred_element_type=jnp.float32)
    asd_ref[...] = asd

    @pl.when(i == 0)
    def _():
        mx_ref[...] = jnp.full_like(mx_ref, -jnp.inf)

    mx_ref[0:1, :] = jnp.maximum(mx_ref[0:1, :], jnp.max(asd, axis=0)[None, :])


def _k5_body(num_ref, den_ref, bias_ref, wc_ref, att_ref,
             xw_ref, asd_ref, mx_ref):
    i = pl.program_id(0)
    den = den_ref[...]
    h = num_ref[...] / (jnp.repeat(den, HID, axis=1) + 1e-16) + bias_ref[...][None, :]
    h = jnp.maximum(h, 0.0)
    xw = jnp.dot(h, wc_ref[...], preferred_element_type=jnp.float32)
    xw_ref[...] = xw
    asd = jnp.dot(xw, att_ref[...], preferred_element_type=jnp.float32)
    asd_ref[...] = asd

    @pl.when(i == 0)
    def _():
        mx_ref[...] = jnp.full_like(mx_ref, -jnp.inf)

    mx_ref[0:1, :] = jnp.maximum(mx_ref[0:1, :], jnp.max(asd, axis=0)[None, :])


def _k4_body(ea_ref, ae_mat_ref, aeo_ref, mx_ref):
    i = pl.program_id(0)
    aeo = jnp.dot(ea_ref[...], ae_mat_ref[...], preferred_element_type=jnp.float32)
    aeo_ref[...] = aeo

    @pl.when(i == 0)
    def _():
        mx_ref[...] = jnp.full_like(mx_ref, -jnp.inf)

    mx_ref[0:1, :] = jnp.maximum(mx_ref[0:1, :], jnp.max(aeo, axis=0)[None, :])


def _k6_body(num_ref, den_ref, bias_ref, out_ref):
    den = den_ref[...]
    out_ref[...] = num_ref[...] / (jnp.repeat(den, OUT, axis=1) + 1e-16) + bias_ref[...][None, :]


def _full_spec(shape):
    return pl.BlockSpec(shape, lambda i: tuple(0 for _ in shape))


def _node_pipeline(x, W1, b1, g1, be1, W2, b2, g_emb, be_emb, Wc1, att1):
    t1, s1 = pl.pallas_call(
        _k1_body,
        grid=(N_GRID,),
        in_specs=[pl.BlockSpec((NB, D_IN), lambda i: (i, 0)),
                  _full_spec((D_IN, HID)), _full_spec((HID,))],
        out_specs=[pl.BlockSpec((NB, HID), lambda i: (i, 0)),
                   pl.BlockSpec((8, HID), lambda i: (0, 0))],
        out_shape=[jax.ShapeDtypeStruct((N, HID), jnp.float32),
                   jax.ShapeDtypeStruct((8, HID), jnp.float32)],
    )(x, W1, b1)
    t2, s2 = pl.pallas_call(
        _k2_body,
        grid=(N_GRID,),
        in_specs=[pl.BlockSpec((NB, HID), lambda i: (i, 0)),
                  _full_spec((8, HID)), _full_spec((HID, HID)),
                  _full_spec((HID,)), _full_spec((HID,)), _full_spec((HID,))],
        out_specs=[pl.BlockSpec((NB, HID), lambda i: (i, 0)),
                   pl.BlockSpec((8, HID), lambda i: (0, 0))],
        out_shape=[jax.ShapeDtypeStruct((N, HID), jnp.float32),
                   jax.ShapeDtypeStruct((8, HID), jnp.float32)],
    )(t1, s1, W2, b2, g1, be1)
    xw1, asd1, mx1 = pl.pallas_call(
        _k3_body,
        grid=(N_GRID,),
        in_specs=[pl.BlockSpec((NB, HID), lambda i: (i, 0)),
                  _full_spec((8, HID)), _full_spec((HID,)), _full_spec((HID,)),
                  _full_spec((HID, 2 * HID)), _full_spec((2 * HID, 2 * HEADS))],
        out_specs=[pl.BlockSpec((NB, 2 * HID), lambda i: (i, 0)),
                   pl.BlockSpec((NB, 2 * HEADS), lambda i: (i, 0)),
                   pl.BlockSpec((8, 2 * HEADS), lambda i: (0, 0))],
        out_shape=[jax.ShapeDtypeStruct((N, 2 * HID), jnp.float32),
                   jax.ShapeDtypeStruct((N, 2 * HEADS), jnp.float32),
                   jax.ShapeDtypeStruct((8, 2 * HEADS), jnp.float32)],
    )(t2, s2, g_emb, be_emb, Wc1, att1)
    return xw1, asd1, mx1


def _att_mat(att_s, att_d, H, C):
    m = jnp.zeros((H * C, 2 * H), jnp.float32)
    for h in range(H):
        m = m.at[h * C:(h + 1) * C, h].set(att_s[h])
        m = m.at[h * C:(h + 1) * C, H + h].set(att_d[h])
    return m


def _edge_phase_jnp(src, dst, xw, asd, a_e, bound, H, C):
    a_s = asd[:, :H]
    a_d = asd[:, H:]
    alpha = a_s[src] + a_d[dst] + a_e
    alpha = jnp.where(alpha >= 0, alpha, 0.2 * alpha)
    ex = jnp.exp(alpha - bound[None, :])
    denom = jax.ops.segment_sum(ex, dst, num_segments=N)
    msg = xw[src].reshape(-1, H, C) * ex[:, :, None]
    num = jax.ops.segment_sum(msg.reshape(-1, H * C), dst, num_segments=N)
    return num, denom


def kernel(x, edge_index, edge_attr, W1, b1, g1, be1, W2, b2, g_emb, be_emb,
           Wc1, as1, ad1, We1, ae1, bias1, Wc2, as2, ad2, We2, ae2, bias2):
    src = edge_index[0].astype(jnp.int32)
    dst = edge_index[1].astype(jnp.int32)

    att1 = _att_mat(as1, ad1, HEADS, HID)
    att2 = _att_mat(as2, ad2, HEADS, OUT)
    Ae1 = (We1.reshape(EDGE_DIM, HEADS, HID) * ae1[None, :, :]).sum(-1)
    Ae2 = (We2.reshape(EDGE_DIM, HEADS, OUT) * ae2[None, :, :]).sum(-1)
    AeAll = jnp.concatenate([Ae1, Ae2], axis=1)

    xw1, asd1, mx1 = _node_pipeline(x, W1, b1, g1, be1, W2, b2, g_emb, be_emb,
                                    Wc1, att1)

    ae_all, mxe = pl.pallas_call(
        _k4_body,
        grid=(E_GRID,),
        in_specs=[pl.BlockSpec((EB, EDGE_DIM), lambda i: (i, 0)),
                  _full_spec((EDGE_DIM, 2 * HEADS))],
        out_specs=[pl.BlockSpec((EB, 2 * HEADS), lambda i: (i, 0)),
                   pl.BlockSpec((8, 2 * HEADS), lambda i: (0, 0))],
        out_shape=[jax.ShapeDtypeStruct((E, 2 * HEADS), jnp.float32),
                   jax.ShapeDtypeStruct((8, 2 * HEADS), jnp.float32)],
    )(edge_attr, AeAll)

    bound1 = mx1[0, :HEADS] + mx1[0, HEADS:] + mxe[0, :HEADS]
    num1, den1 = _edge_phase_jnp(src, dst, xw1, asd1, ae_all[:, :HEADS],
                                 bound1, HEADS, HID)

    xw2, asd2, mx2 = pl.pallas_call(
        _k5_body,
        grid=(N_GRID,),
        in_specs=[pl.BlockSpec((NB, HEADS * HID), lambda i: (i, 0)),
                  pl.BlockSpec((NB, HEADS), lambda i: (i, 0)),
                  _full_spec((HEADS * HID,)),
                  _full_spec((HEADS * HID, HEADS * OUT)),
                  _full_spec((HEADS * OUT, 2 * HEADS))],
        out_specs=[pl.BlockSpec((NB, HEADS * OUT), lambda i: (i, 0)),
                   pl.BlockSpec((NB, 2 * HEADS), lambda i: (i, 0)),
                   pl.BlockSpec((8, 2 * HEADS), lambda i: (0, 0))],
        out_shape=[jax.ShapeDtypeStruct((N, HEADS * OUT), jnp.float32),
                   jax.ShapeDtypeStruct((N, 2 * HEADS), jnp.float32),
                   jax.ShapeDtypeStruct((8, 2 * HEADS), jnp.float32)],
    )(num1, den1, bias1, Wc2, att2)

    bound2 = mx2[0, :HEADS] + mx2[0, HEADS:] + mxe[0, HEADS:]
    num2, den2 = _edge_phase_jnp(src, dst, xw2, asd2, ae_all[:, HEADS:],
                                 bound2, HEADS, OUT)

    out = pl.pallas_call(
        _k6_body,
        grid=(N_GRID,),
        in_specs=[pl.BlockSpec((NB, HEADS * OUT), lambda i: (i, 0)),
                  pl.BlockSpec((NB, HEADS), lambda i: (i, 0)),
                  _full_spec((HEADS * OUT,))],
        out_specs=pl.BlockSpec((NB, HEADS * OUT), lambda i: (i, 0)),
        out_shape=jax.ShapeDtypeStruct((N, HEADS * OUT), jnp.float32),
    )(num2, den2, bias2)
    return out


# speedup vs baseline: 11.4599x; 11.4599x over previous
import jax
import jax.numpy as jnp
from jax.experimental import pallas as pl

N = 100000
E = 1600000
D_IN = 128
HID = 16
HEADS = 2
OUT = 16
EDGE_DIM = 8

NB = 2000
N_GRID = N // NB
EB = 12800
E_GRID = E // EB


def _k1_body(x_ref, w1_ref, b1_ref, t1_ref, s_ref):
    i = pl.program_id(0)
    t1 = jnp.dot(x_ref[...], w1_ref[...], preferred_element_type=jnp.float32)
    t1 = t1 + b1_ref[...][None, :]
    t1_ref[...] = t1

    @pl.when(i == 0)
    def _():
        s_ref[...] = jnp.zeros_like(s_ref)

    s_ref[0:1, :] += jnp.sum(t1, axis=0)[None, :]
    s_ref[1:2, :] += jnp.sum(t1 * t1, axis=0)[None, :]


def _k2_body(t1_ref, s1_ref, w2_ref, b2_ref, g1_ref, be1_ref, t2_ref, s_ref):
    i = pl.program_id(0)
    s1 = s1_ref[...]
    m = s1[0, :] / N
    v = s1[1, :] / N - m * m
    rstd = jax.lax.rsqrt(v + 1e-5)
    h1 = (t1_ref[...] - m[None, :]) * (rstd * g1_ref[...])[None, :] + be1_ref[...][None, :]
    h1 = jnp.maximum(h1, 0.0)
    t2 = jnp.dot(h1, w2_ref[...], preferred_element_type=jnp.float32)
    t2 = t2 + b2_ref[...][None, :]
    t2_ref[...] = t2

    @pl.when(i == 0)
    def _():
        s_ref[...] = jnp.zeros_like(s_ref)

    s_ref[0:1, :] += jnp.sum(t2, axis=0)[None, :]
    s_ref[1:2, :] += jnp.sum(t2 * t2, axis=0)[None, :]


def _k3_body(t2_ref, s2_ref, g_ref, be_ref, wc_ref, att_ref,
             xw_ref, asd_ref, mx_ref):
    i = pl.program_id(0)
    s2 = s2_ref[...]
    m = s2[0, :] / N
    v = s2[1, :] / N - m * m
    rstd = jax.lax.rsqrt(v + 1e-5)
    h = (t2_ref[...] - m[None, :]) * (rstd * g_ref[...])[None, :] + be_ref[...][None, :]
    h = jnp.maximum(h, 0.0)
    xw = jnp.dot(h, wc_ref[...], preferred_element_type=jnp.float32)
    xw_ref[...] = xw
    asd = jnp.dot(xw, att_ref[...], preferred_element_type=jnp.float32)
    asd_ref[...] = asd

    @pl.when(i == 0)
    def _():
        mx_ref[...] = jnp.full_like(mx_ref, -jnp.inf)

    mx_ref[0:1, :] = jnp.maximum(mx_ref[0:1, :], jnp.max(asd, axis=0)[None, :])


def _k5_body(num_ref, den_ref, bias_ref, wc_ref, att_ref,
             xw_ref, asd_ref, mx_ref):
    i = pl.program_id(0)
    den = den_ref[...]
    h = num_ref[...] / (jnp.repeat(den, HID, axis=1) + 1e-16) + bias_ref[...][None, :]
    h = jnp.maximum(h, 0.0)
    xw = jnp.dot(h, wc_ref[...], preferred_element_type=jnp.float32)
    xw_ref[...] = xw
    asd = jnp.dot(xw, att_ref[...], preferred_element_type=jnp.float32)
    asd_ref[...] = asd

    @pl.when(i == 0)
    def _():
        mx_ref[...] = jnp.full_like(mx_ref, -jnp.inf)

    mx_ref[0:1, :] = jnp.maximum(mx_ref[0:1, :], jnp.max(asd, axis=0)[None, :])


def _k4_body(ea_ref, ae_mat_ref, aeo_ref, mx_ref):
    i = pl.program_id(0)
    aeo = jnp.dot(ea_ref[...], ae_mat_ref[...], preferred_element_type=jnp.float32)
    aeo_ref[...] = aeo

    @pl.when(i == 0)
    def _():
        mx_ref[...] = jnp.full_like(mx_ref, -jnp.inf)

    mx_ref[0:1, :] = jnp.maximum(mx_ref[0:1, :], jnp.max(aeo, axis=0)[None, :])


def _k6_body(num_ref, den_ref, bias_ref, out_ref):
    den = den_ref[...]
    out_ref[...] = num_ref[...] / (jnp.repeat(den, OUT, axis=1) + 1e-16) + bias_ref[...][None, :]


def _full_spec(shape):
    return pl.BlockSpec(shape, lambda i: tuple(0 for _ in shape))


def _node_pipeline(x, W1, b1, g1, be1, W2, b2, g_emb, be_emb, Wc1, att1):
    t1, s1 = pl.pallas_call(
        _k1_body,
        grid=(N_GRID,),
        in_specs=[pl.BlockSpec((NB, D_IN), lambda i: (i, 0)),
                  _full_spec((D_IN, HID)), _full_spec((HID,))],
        out_specs=[pl.BlockSpec((NB, HID), lambda i: (i, 0)),
                   pl.BlockSpec((8, HID), lambda i: (0, 0))],
        out_shape=[jax.ShapeDtypeStruct((N, HID), jnp.float32),
                   jax.ShapeDtypeStruct((8, HID), jnp.float32)],
    )(x, W1, b1)
    t2, s2 = pl.pallas_call(
        _k2_body,
        grid=(N_GRID,),
        in_specs=[pl.BlockSpec((NB, HID), lambda i: (i, 0)),
                  _full_spec((8, HID)), _full_spec((HID, HID)),
                  _full_spec((HID,)), _full_spec((HID,)), _full_spec((HID,))],
        out_specs=[pl.BlockSpec((NB, HID), lambda i: (i, 0)),
                   pl.BlockSpec((8, HID), lambda i: (0, 0))],
        out_shape=[jax.ShapeDtypeStruct((N, HID), jnp.float32),
                   jax.ShapeDtypeStruct((8, HID), jnp.float32)],
    )(t1, s1, W2, b2, g1, be1)
    xw1, asd1, mx1 = pl.pallas_call(
        _k3_body,
        grid=(N_GRID,),
        in_specs=[pl.BlockSpec((NB, HID), lambda i: (i, 0)),
                  _full_spec((8, HID)), _full_spec((HID,)), _full_spec((HID,)),
                  _full_spec((HID, 2 * HID)), _full_spec((2 * HID, 2 * HEADS))],
        out_specs=[pl.BlockSpec((NB, 2 * HID), lambda i: (i, 0)),
                   pl.BlockSpec((NB, 2 * HEADS), lambda i: (i, 0)),
                   pl.BlockSpec((8, 2 * HEADS), lambda i: (0, 0))],
        out_shape=[jax.ShapeDtypeStruct((N, 2 * HID), jnp.float32),
                   jax.ShapeDtypeStruct((N, 2 * HEADS), jnp.float32),
                   jax.ShapeDtypeStruct((8, 2 * HEADS), jnp.float32)],
    )(t2, s2, g_emb, be_emb, Wc1, att1)
    return xw1, asd1, mx1


def _att_mat(att_s, att_d, H, C):
    m = jnp.zeros((H * C, 2 * H), jnp.float32)
    for h in range(H):
        m = m.at[h * C:(h + 1) * C, h].set(att_s[h])
        m = m.at[h * C:(h + 1) * C, H + h].set(att_d[h])
    return m


def _edge_phase_jnp(src, dst, xw, asd, a_e, bound, H, C):
    a_s = asd[:, :H]
    a_d = asd[:, H:]
    alpha = a_s[src] + a_d[dst] + a_e
    alpha = jnp.where(alpha >= 0, alpha, 0.2 * alpha)
    ex = jnp.exp(alpha - bound[None, :])
    denom = jax.ops.segment_sum(ex, dst, num_segments=N)
    msg = xw[src].reshape(-1, H, C) * ex[:, :, None]
    num = jax.ops.segment_sum(msg.reshape(-1, H * C), dst, num_segments=N)
    return num, denom


def kernel(x, edge_index, edge_attr, W1, b1, g1, be1, W2, b2, g_emb, be_emb,
           Wc1, as1, ad1, We1, ae1, bias1, Wc2, as2, ad2, We2, ae2, bias2):
    src = edge_index[0].astype(jnp.int32)
    dst = edge_index[1].astype(jnp.int32)

    att1 = _att_mat(as1, ad1, HEADS, HID)
    att2 = _att_mat(as2, ad2, HEADS, OUT)
    Ae1 = (We1.reshape(EDGE_DIM, HEADS, HID) * ae1[None, :, :]).sum(-1)
    Ae2 = (We2.reshape(EDGE_DIM, HEADS, OUT) * ae2[None, :, :]).sum(-1)
    AeAll = jnp.concatenate([Ae1, Ae2], axis=1)

    xw1, asd1, mx1 = _node_pipeline(x, W1, b1, g1, be1, W2, b2, g_emb, be_emb,
                                    Wc1, att1)

    ae_all, mxe = pl.pallas_call(
        _k4_body,
        grid=(E_GRID,),
        in_specs=[pl.BlockSpec((EB, EDGE_DIM), lambda i: (i, 0)),
                  _full_spec((EDGE_DIM, 2 * HEADS))],
        out_specs=[pl.BlockSpec((EB, 2 * HEADS), lambda i: (i, 0)),
                   pl.BlockSpec((8, 2 * HEADS), lambda i: (0, 0))],
        out_shape=[jax.ShapeDtypeStruct((E, 2 * HEADS), jnp.float32),
                   jax.ShapeDtypeStruct((8, 2 * HEADS), jnp.float32)],
    )(edge_attr, AeAll)

    bound1 = mx1[0, :HEADS] + mx1[0, HEADS:] + mxe[0, :HEADS]
    num1, den1 = _edge_phase_jnp(src, dst, xw1, asd1, ae_all[:, :HEADS],
                                 bound1, HEADS, HID)

    xw2, asd2, mx2 = pl.pallas_call(
        _k5_body,
        grid=(N_GRID,),
        in_specs=[pl.BlockSpec((NB, HEADS * HID), lambda i: (i, 0)),
                  pl.BlockSpec((NB, HEADS), lambda i: (i, 0)),
                  _full_spec((HEADS * HID,)),
                  _full_spec((HEADS * HID, HEADS * OUT)),
                  _full_spec((HEADS * OUT, 2 * HEADS))],
        out_specs=[pl.BlockSpec((NB, HEADS * OUT), lambda i: (i, 0)),
                   pl.BlockSpec((NB, 2 * HEADS), lambda i: (i, 0)),
                   pl.BlockSpec((8, 2 * HEADS), lambda i: (0, 0))],
        out_shape=[jax.ShapeDtypeStruct((N, HEADS * OUT), jnp.float32),
                   jax.ShapeDtypeStruct((N, 2 * HEADS), jnp.float32),
                   jax.ShapeDtypeStruct((8, 2 * HEADS), jnp.float32)],
    )(num1, den1, bias1, Wc2, att2)

    bound2 = mx2[0, :HEADS] + mx2[0, HEADS:] + mxe[0, HEADS:]
    num2, den2 = _edge_phase_jnp(src, dst, xw2, asd2, ae_all[:, HEADS:],
                                 bound2, HEADS, OUT)

    out = pl.pallas_call(
        _k6_body,
        grid=(N_GRID,),
        in_specs=[pl.BlockSpec((NB, HEADS * OUT), lambda i: (i, 0)),
                  pl.BlockSpec((NB, HEADS), lambda i: (i, 0)),
                  _full_spec((HEADS * OUT,))],
        out_specs=pl.BlockSpec((NB, HEADS * OUT), lambda i: (i, 0)),
        out_shape=jax.ShapeDtypeStruct((N, HEADS * OUT), jnp.float32),
    )(num2, den2, bias2)
    return out


# speedup vs baseline: 66.9981x; 5.8463x over previous
import functools

import jax
import jax.numpy as jnp
from jax import lax
from jax.experimental import pallas as pl
from jax.experimental.pallas import tpu as pltpu
from jax.experimental.pallas import tpu_sc as plsc

N = 100000
E = 1600000
D_IN = 128
HID = 16
HEADS = 2
OUT = 16
EDGE_DIM = 8

NB = 2000
N_GRID = N // NB
EB = 12800
E_GRID = E // EB

ECP = 12800
EP = ECP * 128
CR = 5
KE = CR * 128
NCHUNK = ECP // CR
WB = 400
NWB = N // WB
SWB = 1000
NSWB = N // SWB


def _k1_body(x_ref, w1_ref, b1_ref, t1_ref, s_ref):
    i = pl.program_id(0)
    t1 = jnp.dot(x_ref[...], w1_ref[...], preferred_element_type=jnp.float32)
    t1 = t1 + b1_ref[...][None, :]
    t1_ref[...] = t1

    @pl.when(i == 0)
    def _():
        s_ref[...] = jnp.zeros_like(s_ref)

    s_ref[0:1, :] += jnp.sum(t1, axis=0)[None, :]
    s_ref[1:2, :] += jnp.sum(t1 * t1, axis=0)[None, :]


def _k2_body(t1_ref, s1_ref, w2_ref, b2_ref, g1_ref, be1_ref, t2_ref, s_ref):
    i = pl.program_id(0)
    s1 = s1_ref[...]
    m = s1[0, :] / N
    v = s1[1, :] / N - m * m
    rstd = jax.lax.rsqrt(v + 1e-5)
    h1 = (t1_ref[...] - m[None, :]) * (rstd * g1_ref[...])[None, :] + be1_ref[...][None, :]
    h1 = jnp.maximum(h1, 0.0)
    t2 = jnp.dot(h1, w2_ref[...], preferred_element_type=jnp.float32)
    t2 = t2 + b2_ref[...][None, :]
    t2_ref[...] = t2

    @pl.when(i == 0)
    def _():
        s_ref[...] = jnp.zeros_like(s_ref)

    s_ref[0:1, :] += jnp.sum(t2, axis=0)[None, :]
    s_ref[1:2, :] += jnp.sum(t2 * t2, axis=0)[None, :]


def _k3_body(t2_ref, s2_ref, g_ref, be_ref, wc_ref, att_ref,
             xw_ref, asd_ref, mx_ref):
    i = pl.program_id(0)
    s2 = s2_ref[...]
    m = s2[0, :] / N
    v = s2[1, :] / N - m * m
    rstd = jax.lax.rsqrt(v + 1e-5)
    h = (t2_ref[...] - m[None, :]) * (rstd * g_ref[...])[None, :] + be_ref[...][None, :]
    h = jnp.maximum(h, 0.0)
    xw = jnp.dot(h, wc_ref[...], preferred_element_type=jnp.float32)
    xw_ref[0] = xw[:, :HID]
    xw_ref[1] = xw[:, HID:]
    asd = jnp.dot(xw, att_ref[...], preferred_element_type=jnp.float32)
    asd_ref[...] = asd

    @pl.when(i == 0)
    def _():
        mx_ref[...] = jnp.full_like(mx_ref, -jnp.inf)

    mx_ref[0:1, :] = jnp.maximum(mx_ref[0:1, :], jnp.max(asd, axis=0)[None, :])


def _k5_body(num_ref, den_ref, bias_ref, wc_ref, att_ref,
             xw_ref, asd_ref, mx_ref):
    i = pl.program_id(0)
    h0 = num_ref[0] / (den_ref[0] + 1e-16)
    h1 = num_ref[1] / (den_ref[1] + 1e-16)
    h = jnp.concatenate([h0, h1], axis=1) + bias_ref[...][None, :]
    h = jnp.maximum(h, 0.0)
    xw = jnp.dot(h, wc_ref[...], preferred_element_type=jnp.float32)
    xw_ref[0] = xw[:, :OUT]
    xw_ref[1] = xw[:, OUT:]
    asd = jnp.dot(xw, att_ref[...], preferred_element_type=jnp.float32)
    asd_ref[...] = asd

    @pl.when(i == 0)
    def _():
        mx_ref[...] = jnp.full_like(mx_ref, -jnp.inf)

    mx_ref[0:1, :] = jnp.maximum(mx_ref[0:1, :], jnp.max(asd, axis=0)[None, :])


def _k4_body(ea_ref, ae_mat_ref, aeo_ref, mx_ref):
    i = pl.program_id(0)
    aeo = jnp.dot(ea_ref[...], ae_mat_ref[...], preferred_element_type=jnp.float32)
    aeo_ref[...] = aeo

    @pl.when(i == 0)
    def _():
        mx_ref[...] = jnp.full_like(mx_ref, -jnp.inf)

    mx_ref[0:1, :] = jnp.maximum(mx_ref[0:1, :], jnp.max(aeo, axis=0)[None, :])


def _k6_body(num_ref, den_ref, bias_ref, out_ref):
    h0 = num_ref[0] / (den_ref[0] + 1e-16)
    h1 = num_ref[1] / (den_ref[1] + 1e-16)
    out_ref[...] = jnp.concatenate([h0, h1], axis=1) + bias_ref[...][None, :]


def _full_spec(shape):
    return pl.BlockSpec(shape, lambda i: tuple(0 for _ in shape))


def _node_pipeline(x, W1, b1, g1, be1, W2, b2, g_emb, be_emb, Wc1, att1):
    t1, s1 = pl.pallas_call(
        _k1_body,
        grid=(N_GRID,),
        in_specs=[pl.BlockSpec((NB, D_IN), lambda i: (i, 0)),
                  _full_spec((D_IN, HID)), _full_spec((HID,))],
        out_specs=[pl.BlockSpec((NB, HID), lambda i: (i, 0)),
                   pl.BlockSpec((8, HID), lambda i: (0, 0))],
        out_shape=[jax.ShapeDtypeStruct((N, HID), jnp.float32),
                   jax.ShapeDtypeStruct((8, HID), jnp.float32)],
    )(x, W1, b1)
    t2, s2 = pl.pallas_call(
        _k2_body,
        grid=(N_GRID,),
        in_specs=[pl.BlockSpec((NB, HID), lambda i: (i, 0)),
                  _full_spec((8, HID)), _full_spec((HID, HID)),
                  _full_spec((HID,)), _full_spec((HID,)), _full_spec((HID,))],
        out_specs=[pl.BlockSpec((NB, HID), lambda i: (i, 0)),
                   pl.BlockSpec((8, HID), lambda i: (0, 0))],
        out_shape=[jax.ShapeDtypeStruct((N, HID), jnp.float32),
                   jax.ShapeDtypeStruct((8, HID), jnp.float32)],
    )(t1, s1, W2, b2, g1, be1)
    xwh, asd, mx = pl.pallas_call(
        _k3_body,
        grid=(N_GRID,),
        in_specs=[pl.BlockSpec((NB, HID), lambda i: (i, 0)),
                  _full_spec((8, HID)), _full_spec((HID,)), _full_spec((HID,)),
                  _full_spec((HID, 2 * HID)), _full_spec((2 * HID, 2 * HEADS))],
        out_specs=[pl.BlockSpec((2, NB, HID), lambda i: (0, i, 0)),
                   pl.BlockSpec((NB, 2 * HEADS), lambda i: (i, 0)),
                   pl.BlockSpec((8, 2 * HEADS), lambda i: (0, 0))],
        out_shape=[jax.ShapeDtypeStruct((2, N, HID), jnp.float32),
                   jax.ShapeDtypeStruct((N, 2 * HEADS), jnp.float32),
                   jax.ShapeDtypeStruct((8, 2 * HEADS), jnp.float32)],
    )(t2, s2, g_emb, be_emb, Wc1, att1)
    return xwh, asd, mx


def _att_mat(att_s, att_d, H, C):
    m = jnp.zeros((H * C, 2 * H), jnp.float32)
    for h in range(H):
        m = m.at[h * C:(h + 1) * C, h].set(att_s[h])
        m = m.at[h * C:(h + 1) * C, H + h].set(att_d[h])
    return m


def _sc_conv_body(xw_hbm, src_hbm, dstr_hbm, ae_hbm, bnd_hbm,
                  as_hbm, ad_hbm,
                  num0_out, num1_out, den0_out, den1_out,
                  num_sp, den_sp,
                  srcb, dstb, dstfb, aeb, asb, adb, exb, msg, bnd,
                  sem_in, sem_g, sem_s):
    c = lax.axis_index("c")
    s = lax.axis_index("s")

    pltpu.sync_copy(bnd_hbm.at[pl.ds(c * 16, 16)], bnd)

    @pl.loop(0, KE)
    def _(r):
        msg[r, :] = jnp.zeros((HID,), jnp.float32)

    @pl.loop(0, (KE + 16) // 16)
    def _(r):
        exb[pl.ds(r * 16, 16)] = jnp.zeros((16,), jnp.float32)

    @pl.loop(s, NWB, step=16)
    def _(i):
        sl = pl.ds(i * WB, WB)
        pltpu.sync_copy(msg.at[pl.ds(0, WB)], num_sp.at[sl])
        pltpu.sync_copy(exb.at[pl.ds(0, WB)], den_sp.at[sl])

    plsc.subcore_barrier()

    bndv = bnd[...]
    cn = c * N

    @pl.loop(s, NCHUNK, step=16)
    def _(i):
        r0 = i * CR
        e0 = i * KE
        cp0 = pltpu.async_copy(src_hbm.at[pl.ds(e0, KE)], srcb, sem_in)
        cp1 = pltpu.async_copy(dstr_hbm.at[pl.ds(r0, CR)], dstb, sem_in)
        cp3 = pltpu.async_copy(ae_hbm.at[pl.ds(c * EP + e0, KE)], aeb, sem_in)
        cp0.wait()
        cp1.wait()
        cp3.wait()

        @pl.loop(0, KE // 16)
        def _(t):
            sl = pl.ds(t * 16, 16)
            srcb[sl] = srcb[sl] + cn
        for j in range(CR):
            for k in range(8):
                dstfb[pl.ds((j * 8 + k) * 16, 16)] = (
                    dstb[j, pl.ds(k * 16, 16)] + cn)

        g = [pltpu.async_copy(as_hbm.at[srcb.at[pl.ds(j * 128, 128)]],
                              asb.at[pl.ds(j * 128, 128)], sem_g)
             for j in range(CR)]
        g += [pltpu.async_copy(ad_hbm.at[dstfb.at[pl.ds(j * 128, 128)]],
                               adb.at[pl.ds(j * 128, 128)], sem_g)
              for j in range(CR)]
        g += [pltpu.async_copy(xw_hbm.at[srcb.at[pl.ds(j * 128, 128)]],
                               msg.at[pl.ds(j * 128, 128)], sem_g)
              for j in range(CR)]
        for cp in g:
            cp.wait()

        @pl.loop(0, KE // 16)
        def _(t):
            sl = pl.ds(t * 16, 16)
            a = asb[sl] + adb[sl] + aeb[sl]
            lr = jnp.maximum(a, 0.2 * a)
            exb[sl] = jnp.exp(lr - bndv)

        @pl.loop(0, KE // 16)
        def _(g_):
            exv = exb[pl.ds(g_ * 16, 16)]
            base = g_ * 16
            for k in range(16):
                msg[base + k, :] = msg[base + k, :] * exv[k]

        sc = []
        for j in range(CR):
            sc.append(pltpu.async_copy(msg.at[pl.ds(j * 128, 128)],
                                       num_sp.at[dstb.at[j]], sem_s, add=True))
            sc.append(pltpu.async_copy(exb.at[pl.ds(j * 128, 128)],
                                       den_sp.at[dstb.at[j]], sem_s, add=True))
        for cp in sc:
            cp.wait()

    plsc.subcore_barrier()

    @pl.loop(s, NWB, step=16)
    def _(i):
        sl = pl.ds(i * WB, WB)
        pltpu.sync_copy(num_sp.at[sl], msg.at[pl.ds(0, WB)])
        pltpu.sync_copy(den_sp.at[sl], exb.at[pl.ds(0, WB)])

        @pl.when(c == 0)
        def _():
            pltpu.sync_copy(msg.at[pl.ds(0, WB)], num0_out.at[sl])
            pltpu.sync_copy(exb.at[pl.ds(0, WB)], den0_out.at[sl])

        @pl.when(c == 1)
        def _():
            pltpu.sync_copy(msg.at[pl.ds(0, WB)], num1_out.at[sl])
            pltpu.sync_copy(exb.at[pl.ds(0, WB)], den1_out.at[sl])


def _sc_conv(xwflat, src_flat, dst_rows, ae_flat, as_flat, ad_flat, bounds):
    mesh = plsc.VectorSubcoreMesh(core_axis_name="c", subcore_axis_name="s")
    return pl.kernel(
        _sc_conv_body,
        out_type=[jax.ShapeDtypeStruct((N, HID), jnp.float32),
                  jax.ShapeDtypeStruct((N, HID), jnp.float32),
                  jax.ShapeDtypeStruct((N,), jnp.float32),
                  jax.ShapeDtypeStruct((N,), jnp.float32)],
        mesh=mesh,
        compiler_params=pltpu.CompilerParams(use_tc_tiling_on_sc=False),
        scratch_types=[
            pltpu.VMEM_SHARED((N, HID), jnp.float32),
            pltpu.VMEM_SHARED((N,), jnp.float32),
            pltpu.VMEM((KE,), jnp.int32),
            pltpu.VMEM((CR, 128), jnp.int32),
            pltpu.VMEM((KE,), jnp.int32),
            pltpu.VMEM((KE,), jnp.float32),
            pltpu.VMEM((KE,), jnp.float32),
            pltpu.VMEM((KE,), jnp.float32),
            pltpu.VMEM((KE + 16,), jnp.float32),
            pltpu.VMEM((KE, HID), jnp.float32),
            pltpu.VMEM((16,), jnp.float32),
            pltpu.SemaphoreType.DMA,
            pltpu.SemaphoreType.DMA,
            pltpu.SemaphoreType.DMA,
        ],
    )(xwflat, src_flat, dst_rows, ae_flat, bounds, as_flat, ad_flat)


def kernel(x, edge_index, edge_attr, W1, b1, g1, be1, W2, b2, g_emb, be_emb,
           Wc1, as1, ad1, We1, ae1, bias1, Wc2, as2, ad2, We2, ae2, bias2):
    src = edge_index[0].astype(jnp.int32)
    dst = edge_index[1].astype(jnp.int32)
    src_p = jnp.concatenate([src, jnp.zeros((EP - E,), jnp.int32)])
    dst_p = jnp.concatenate([dst, jnp.zeros((EP - E,), jnp.int32)])
    dst_rows = dst_p.reshape(ECP, 128)

    att1 = _att_mat(as1, ad1, HEADS, HID)
    att2 = _att_mat(as2, ad2, HEADS, OUT)
    Ae1 = (We1.reshape(EDGE_DIM, HEADS, HID) * ae1[None, :, :]).sum(-1)
    Ae2 = (We2.reshape(EDGE_DIM, HEADS, OUT) * ae2[None, :, :]).sum(-1)
    AeAll = jnp.concatenate([Ae1, Ae2], axis=1)

    xwh1, asd1, mx1 = _node_pipeline(x, W1, b1, g1, be1, W2, b2,
                                     g_emb, be_emb, Wc1, att1)

    ae_all, mxe = pl.pallas_call(
        _k4_body,
        grid=(E_GRID,),
        in_specs=[pl.BlockSpec((EB, EDGE_DIM), lambda i: (i, 0)),
                  _full_spec((EDGE_DIM, 2 * HEADS))],
        out_specs=[pl.BlockSpec((EB, 2 * HEADS), lambda i: (i, 0)),
                   pl.BlockSpec((8, 2 * HEADS), lambda i: (0, 0))],
        out_shape=[jax.ShapeDtypeStruct((E, 2 * HEADS), jnp.float32),
                   jax.ShapeDtypeStruct((8, 2 * HEADS), jnp.float32)],
    )(edge_attr, AeAll)
    aeT = ae_all.T
    zpad = jnp.full((EP - E,), -jnp.inf, jnp.float32)
    ae_f1 = jnp.concatenate([aeT[0], zpad, aeT[1], zpad])
    ae_f2 = jnp.concatenate([aeT[2], zpad, aeT[3], zpad])

    bounds1 = jnp.concatenate([
        jnp.broadcast_to(mx1[0, 0] + mx1[0, HEADS] + mxe[0, 0], (16,)),
        jnp.broadcast_to(mx1[0, 1] + mx1[0, HEADS + 1] + mxe[0, 1], (16,))])
    as1_flat = jnp.concatenate([asd1[:, 0], asd1[:, 1]])
    ad1_flat = jnp.concatenate([asd1[:, 2], asd1[:, 3]])
    num10, num11, den10, den11 = _sc_conv(
        xwh1.reshape(2 * N, HID), src_p, dst_rows, ae_f1,
        as1_flat, ad1_flat, bounds1)
    num1 = jnp.stack([num10, num11])
    den1 = jnp.stack([den10, den11])

    xwh2, asd2, mx2 = pl.pallas_call(
        _k5_body,
        grid=(N_GRID,),
        in_specs=[pl.BlockSpec((2, NB, HID), lambda i: (0, i, 0)),
                  pl.BlockSpec((2, NB, 1), lambda i: (0, i, 0)),
                  _full_spec((HEADS * HID,)),
                  _full_spec((HEADS * HID, HEADS * OUT)),
                  _full_spec((HEADS * OUT, 2 * HEADS))],
        out_specs=[pl.BlockSpec((2, NB, OUT), lambda i: (0, i, 0)),
                   pl.BlockSpec((NB, 2 * HEADS), lambda i: (i, 0)),
                   pl.BlockSpec((8, 2 * HEADS), lambda i: (0, 0))],
        out_shape=[jax.ShapeDtypeStruct((2, N, OUT), jnp.float32),
                   jax.ShapeDtypeStruct((N, 2 * HEADS), jnp.float32),
                   jax.ShapeDtypeStruct((8, 2 * HEADS), jnp.float32)],
    )(num1, den1.reshape(2, N, 1), bias1, Wc2, att2)

    bounds2 = jnp.concatenate([
        jnp.broadcast_to(mx2[0, 0] + mx2[0, HEADS] + mxe[0, HEADS], (16,)),
        jnp.broadcast_to(mx2[0, 1] + mx2[0, HEADS + 1] + mxe[0, HEADS + 1], (16,))])
    as2_flat = jnp.concatenate([asd2[:, 0], asd2[:, 1]])
    ad2_flat = jnp.concatenate([asd2[:, 2], asd2[:, 3]])
    num20, num21, den20, den21 = _sc_conv(
        xwh2.reshape(2 * N, OUT), src_p, dst_rows, ae_f2,
        as2_flat, ad2_flat, bounds2)
    num2 = jnp.stack([num20, num21])
    den2 = jnp.stack([den20, den21])

    out = pl.pallas_call(
        _k6_body,
        grid=(N_GRID,),
        in_specs=[pl.BlockSpec((2, NB, OUT), lambda i: (0, i, 0)),
                  pl.BlockSpec((2, NB, 1), lambda i: (0, i, 0)),
                  _full_spec((HEADS * OUT,))],
        out_specs=pl.BlockSpec((NB, HEADS * OUT), lambda i: (i, 0)),
        out_shape=jax.ShapeDtypeStruct((N, HEADS * OUT), jnp.float32),
    )(num2, den2.reshape(2, N, 1), bias2)
    return out


# speedup vs baseline: 97.6065x; 1.4569x over previous
import functools

import jax
import jax.numpy as jnp
from jax import lax
from jax.experimental import pallas as pl
from jax.experimental.pallas import tpu as pltpu
from jax.experimental.pallas import tpu_sc as plsc

N = 100000
E = 1600000
D_IN = 128
HID = 16
HEADS = 2
OUT = 16
EDGE_DIM = 8

NB = 2000
N_GRID = N // NB
EB = 12800
E_GRID = E // EB

ECP = 12800
EP = ECP * 128
CR = 5
KE = CR * 128
NCHUNK = ECP // CR
WB = 400
NWB = N // WB
SWB = 1000
NSWB = N // SWB


def _k1_body(x_ref, w1_ref, b1_ref, t1_ref, s_ref):
    i = pl.program_id(0)
    t1 = jnp.dot(x_ref[...], w1_ref[...], preferred_element_type=jnp.float32)
    t1 = t1 + b1_ref[...][None, :]
    t1_ref[...] = t1

    @pl.when(i == 0)
    def _():
        s_ref[...] = jnp.zeros_like(s_ref)

    s_ref[0:1, :] += jnp.sum(t1, axis=0)[None, :]
    s_ref[1:2, :] += jnp.sum(t1 * t1, axis=0)[None, :]


def _k2_body(t1_ref, s1_ref, w2_ref, b2_ref, g1_ref, be1_ref, t2_ref, s_ref):
    i = pl.program_id(0)
    s1 = s1_ref[...]
    m = s1[0, :] / N
    v = s1[1, :] / N - m * m
    rstd = jax.lax.rsqrt(v + 1e-5)
    h1 = (t1_ref[...] - m[None, :]) * (rstd * g1_ref[...])[None, :] + be1_ref[...][None, :]
    h1 = jnp.maximum(h1, 0.0)
    t2 = jnp.dot(h1, w2_ref[...], preferred_element_type=jnp.float32)
    t2 = t2 + b2_ref[...][None, :]
    t2_ref[...] = t2

    @pl.when(i == 0)
    def _():
        s_ref[...] = jnp.zeros_like(s_ref)

    s_ref[0:1, :] += jnp.sum(t2, axis=0)[None, :]
    s_ref[1:2, :] += jnp.sum(t2 * t2, axis=0)[None, :]


def _k3_body(t2_ref, s2_ref, g_ref, be_ref, wc_ref, att_ref,
             xw_ref, asd_ref, mx_ref):
    i = pl.program_id(0)
    s2 = s2_ref[...]
    m = s2[0, :] / N
    v = s2[1, :] / N - m * m
    rstd = jax.lax.rsqrt(v + 1e-5)
    h = (t2_ref[...] - m[None, :]) * (rstd * g_ref[...])[None, :] + be_ref[...][None, :]
    h = jnp.maximum(h, 0.0)
    xw = jnp.dot(h, wc_ref[...], preferred_element_type=jnp.float32)
    xw_ref[0] = xw[:, :HID]
    xw_ref[1] = xw[:, HID:]
    asd = jnp.dot(xw, att_ref[...], preferred_element_type=jnp.float32)
    asd_ref[...] = asd

    @pl.when(i == 0)
    def _():
        mx_ref[...] = jnp.full_like(mx_ref, -jnp.inf)

    mx_ref[0:1, :] = jnp.maximum(mx_ref[0:1, :], jnp.max(asd, axis=0)[None, :])


def _k5_body(num0_ref, num1_ref, den0_ref, den1_ref, bias_ref, wc_ref,
             att_ref, xw_ref, asd_ref, mx_ref):
    i = pl.program_id(0)
    h0 = num0_ref[...] / (den0_ref[...] + 1e-16)
    h1 = num1_ref[...] / (den1_ref[...] + 1e-16)
    h = jnp.concatenate([h0, h1], axis=1) + bias_ref[...][None, :]
    h = jnp.maximum(h, 0.0)
    xw = jnp.dot(h, wc_ref[...], preferred_element_type=jnp.float32)
    xw_ref[0] = xw[:, :OUT]
    xw_ref[1] = xw[:, OUT:]
    asd = jnp.dot(xw, att_ref[...], preferred_element_type=jnp.float32)
    asd_ref[...] = asd

    @pl.when(i == 0)
    def _():
        mx_ref[...] = jnp.full_like(mx_ref, -jnp.inf)

    mx_ref[0:1, :] = jnp.maximum(mx_ref[0:1, :], jnp.max(asd, axis=0)[None, :])


def _k4_body(ea_ref, ae_matt_ref, aeo_ref, mx_ref):
    i = pl.program_id(0)
    aeo = jax.lax.dot_general(ae_matt_ref[...], ea_ref[...],
                              (((1,), (1,)), ((), ())),
                              preferred_element_type=jnp.float32)

    @pl.when(i == 0)
    def _():
        mx_ref[...] = jnp.full_like(mx_ref, -jnp.inf)

    @pl.when(i < E_GRID)
    def _():
        aeo_ref[...] = aeo
        mx_ref[0:1, :] = jnp.maximum(mx_ref[0:1, :],
                                     jnp.max(aeo, axis=1)[None, :])

    @pl.when(i >= E_GRID)
    def _():
        aeo_ref[...] = jnp.full_like(aeo_ref, -jnp.inf)


def _k6_body(num0_ref, num1_ref, den0_ref, den1_ref, bias_ref, out_ref):
    h0 = num0_ref[...] / (den0_ref[...] + 1e-16)
    h1 = num1_ref[...] / (den1_ref[...] + 1e-16)
    out_ref[...] = jnp.concatenate([h0, h1], axis=1) + bias_ref[...][None, :]


def _full_spec(shape):
    return pl.BlockSpec(shape, lambda i: tuple(0 for _ in shape))


def _node_pipeline(x, W1, b1, g1, be1, W2, b2, g_emb, be_emb, Wc1, att1):
    t1, s1 = pl.pallas_call(
        _k1_body,
        grid=(N_GRID,),
        in_specs=[pl.BlockSpec((NB, D_IN), lambda i: (i, 0)),
                  _full_spec((D_IN, HID)), _full_spec((HID,))],
        out_specs=[pl.BlockSpec((NB, HID), lambda i: (i, 0)),
                   pl.BlockSpec((8, HID), lambda i: (0, 0))],
        out_shape=[jax.ShapeDtypeStruct((N, HID), jnp.float32),
                   jax.ShapeDtypeStruct((8, HID), jnp.float32)],
    )(x, W1, b1)
    t2, s2 = pl.pallas_call(
        _k2_body,
        grid=(N_GRID,),
        in_specs=[pl.BlockSpec((NB, HID), lambda i: (i, 0)),
                  _full_spec((8, HID)), _full_spec((HID, HID)),
                  _full_spec((HID,)), _full_spec((HID,)), _full_spec((HID,))],
        out_specs=[pl.BlockSpec((NB, HID), lambda i: (i, 0)),
                   pl.BlockSpec((8, HID), lambda i: (0, 0))],
        out_shape=[jax.ShapeDtypeStruct((N, HID), jnp.float32),
                   jax.ShapeDtypeStruct((8, HID), jnp.float32)],
    )(t1, s1, W2, b2, g1, be1)
    xwh, asd, mx = pl.pallas_call(
        _k3_body,
        grid=(N_GRID,),
        in_specs=[pl.BlockSpec((NB, HID), lambda i: (i, 0)),
                  _full_spec((8, HID)), _full_spec((HID,)), _full_spec((HID,)),
                  _full_spec((HID, 2 * HID)), _full_spec((2 * HID, 2 * HEADS))],
        out_specs=[pl.BlockSpec((2, NB, HID), lambda i: (0, i, 0)),
                   pl.BlockSpec((NB, 2 * HEADS), lambda i: (i, 0)),
                   pl.BlockSpec((8, 2 * HEADS), lambda i: (0, 0))],
        out_shape=[jax.ShapeDtypeStruct((2, N, HID), jnp.float32),
                   jax.ShapeDtypeStruct((N, 2 * HEADS), jnp.float32),
                   jax.ShapeDtypeStruct((8, 2 * HEADS), jnp.float32)],
    )(t2, s2, g_emb, be_emb, Wc1, att1)
    return xwh, asd, mx


def _att_mat(att_s, att_d, H, C):
    m = jnp.zeros((H * C, 2 * H), jnp.float32)
    for h in range(H):
        m = m.at[h * C:(h + 1) * C, h].set(att_s[h])
        m = m.at[h * C:(h + 1) * C, H + h].set(att_d[h])
    return m


def _sc_conv_body(base, xw_hbm, src_hbm, dstr_hbm, ae_hbm, bnd_hbm,
                  as_hbm, ad_hbm,
                  num0_out, num1_out, den0_out, den1_out,
                  num_sp, den_sp,
                  srcb, dstb, dstfb, aeb, asb, adb, exb, msg, bnd,
                  sem_in, sem_g, sem_s):
    c = lax.axis_index("c")
    s = lax.axis_index("s")

    pltpu.sync_copy(bnd_hbm.at[pl.ds(c * 16, 16)], bnd)

    @pl.loop(0, KE)
    def _(r):
        msg[r, :] = jnp.zeros((HID,), jnp.float32)

    @pl.loop(0, (KE + 16) // 16)
    def _(r):
        exb[pl.ds(r * 16, 16)] = jnp.zeros((16,), jnp.float32)

    @pl.loop(s, NWB, step=16)
    def _(i):
        sl = pl.ds(i * WB, WB)
        pltpu.sync_copy(msg.at[pl.ds(0, WB)], num_sp.at[sl])
        pltpu.sync_copy(exb.at[pl.ds(0, WB)], den_sp.at[sl])

    plsc.subcore_barrier()

    bndv = bnd[...]
    cn = c * N

    @pl.loop(s, NCHUNK, step=16)
    def _(i):
        r0 = i * CR
        e0 = i * KE
        cp0 = pltpu.async_copy(src_hbm.at[pl.ds(e0, KE)], srcb, sem_in)
        cp1 = pltpu.async_copy(dstr_hbm.at[pl.ds(r0, CR)], dstb, sem_in)
        cp3 = pltpu.async_copy(ae_hbm.at[base + c, pl.ds(e0, KE)], aeb,
                               sem_in)
        cp0.wait()
        cp1.wait()
        cp3.wait()

        @pl.loop(0, KE // 16)
        def _(t):
            sl = pl.ds(t * 16, 16)
            srcb[sl] = srcb[sl] + cn
        for j in range(CR):
            for k in range(8):
                dstfb[pl.ds((j * 8 + k) * 16, 16)] = (
                    dstb[j, pl.ds(k * 16, 16)] + cn)

        g = [pltpu.async_copy(as_hbm.at[srcb.at[pl.ds(j * 128, 128)]],
                              asb.at[pl.ds(j * 128, 128)], sem_g)
             for j in range(CR)]
        g += [pltpu.async_copy(ad_hbm.at[dstfb.at[pl.ds(j * 128, 128)]],
                               adb.at[pl.ds(j * 128, 128)], sem_g)
              for j in range(CR)]
        g += [pltpu.async_copy(xw_hbm.at[srcb.at[pl.ds(j * 128, 128)]],
                               msg.at[pl.ds(j * 128, 128)], sem_g)
              for j in range(CR)]
        for cp in g:
            cp.wait()

        @pl.loop(0, KE // 16)
        def _(t):
            sl = pl.ds(t * 16, 16)
            a = asb[sl] + adb[sl] + aeb[sl]
            lr = jnp.maximum(a, 0.2 * a)
            exb[sl] = jnp.exp(lr - bndv)

        @pl.loop(0, KE // 16)
        def _(g_):
            exv = exb[pl.ds(g_ * 16, 16)]
            base = g_ * 16
            for k in range(16):
                msg[base + k, :] = msg[base + k, :] * exv[k]

        sc = []
        for j in range(CR):
            sc.append(pltpu.async_copy(msg.at[pl.ds(j * 128, 128)],
                                       num_sp.at[dstb.at[j]], sem_s, add=True))
            sc.append(pltpu.async_copy(exb.at[pl.ds(j * 128, 128)],
                                       den_sp.at[dstb.at[j]], sem_s, add=True))
        for cp in sc:
            cp.wait()

    plsc.subcore_barrier()

    @pl.loop(s, NWB, step=16)
    def _(i):
        sl = pl.ds(i * WB, WB)
        pltpu.sync_copy(num_sp.at[sl], msg.at[pl.ds(0, WB)])
        pltpu.sync_copy(den_sp.at[sl], exb.at[pl.ds(0, WB)])

        @pl.when(c == 0)
        def _():
            pltpu.sync_copy(msg.at[pl.ds(0, WB)], num0_out.at[sl])
            pltpu.sync_copy(exb.at[pl.ds(0, WB)], den0_out.at[sl])

        @pl.when(c == 1)
        def _():
            pltpu.sync_copy(msg.at[pl.ds(0, WB)], num1_out.at[sl])
            pltpu.sync_copy(exb.at[pl.ds(0, WB)], den1_out.at[sl])


def _sc_conv(base, xwflat, src_flat, dst_rows, aeP, as_flat, ad_flat, bounds):
    mesh = plsc.VectorSubcoreMesh(core_axis_name="c", subcore_axis_name="s")
    return pl.kernel(
        functools.partial(_sc_conv_body, base),
        out_type=[jax.ShapeDtypeStruct((N, HID), jnp.float32),
                  jax.ShapeDtypeStruct((N, HID), jnp.float32),
                  jax.ShapeDtypeStruct((N,), jnp.float32),
                  jax.ShapeDtypeStruct((N,), jnp.float32)],
        mesh=mesh,
        compiler_params=pltpu.CompilerParams(use_tc_tiling_on_sc=False),
        scratch_types=[
            pltpu.VMEM_SHARED((N, HID), jnp.float32),
            pltpu.VMEM_SHARED((N,), jnp.float32),
            pltpu.VMEM((KE,), jnp.int32),
            pltpu.VMEM((CR, 128), jnp.int32),
            pltpu.VMEM((KE,), jnp.int32),
            pltpu.VMEM((KE,), jnp.float32),
            pltpu.VMEM((KE,), jnp.float32),
            pltpu.VMEM((KE,), jnp.float32),
            pltpu.VMEM((KE + 16,), jnp.float32),
            pltpu.VMEM((KE, HID), jnp.float32),
            pltpu.VMEM((16,), jnp.float32),
            pltpu.SemaphoreType.DMA,
            pltpu.SemaphoreType.DMA,
            pltpu.SemaphoreType.DMA,
        ],
    )(xwflat, src_flat, dst_rows, aeP, bounds, as_flat, ad_flat)


def kernel(x, edge_index, edge_attr, W1, b1, g1, be1, W2, b2, g_emb, be_emb,
           Wc1, as1, ad1, We1, ae1, bias1, Wc2, as2, ad2, We2, ae2, bias2):
    src = edge_index[0].astype(jnp.int32)
    dst = edge_index[1].astype(jnp.int32)
    src_p = jnp.concatenate([src, jnp.zeros((EP - E,), jnp.int32)])
    dst_p = jnp.concatenate([dst, jnp.zeros((EP - E,), jnp.int32)])
    dst_rows = dst_p.reshape(ECP, 128)

    att1 = _att_mat(as1, ad1, HEADS, HID)
    att2 = _att_mat(as2, ad2, HEADS, OUT)
    Ae1 = (We1.reshape(EDGE_DIM, HEADS, HID) * ae1[None, :, :]).sum(-1)
    Ae2 = (We2.reshape(EDGE_DIM, HEADS, OUT) * ae2[None, :, :]).sum(-1)
    AeAll = jnp.concatenate([Ae1, Ae2], axis=1)

    xwh1, asd1, mx1 = _node_pipeline(x, W1, b1, g1, be1, W2, b2,
                                     g_emb, be_emb, Wc1, att1)

    aeP, mxe = pl.pallas_call(
        _k4_body,
        grid=(EP // EB,),
        in_specs=[pl.BlockSpec((EB, EDGE_DIM),
                               lambda i: (jnp.minimum(i, E_GRID - 1), 0)),
                  _full_spec((2 * HEADS, EDGE_DIM))],
        out_specs=[pl.BlockSpec((2 * HEADS, EB), lambda i: (0, i)),
                   pl.BlockSpec((8, 2 * HEADS), lambda i: (0, 0))],
        out_shape=[jax.ShapeDtypeStruct((2 * HEADS, EP), jnp.float32),
                   jax.ShapeDtypeStruct((8, 2 * HEADS), jnp.float32)],
    )(edge_attr, AeAll.T)

    bounds1 = jnp.concatenate([
        jnp.broadcast_to(mx1[0, 0] + mx1[0, HEADS] + mxe[0, 0], (16,)),
        jnp.broadcast_to(mx1[0, 1] + mx1[0, HEADS + 1] + mxe[0, 1], (16,))])
    as1_flat = jnp.concatenate([asd1[:, 0], asd1[:, 1]])
    ad1_flat = jnp.concatenate([asd1[:, 2], asd1[:, 3]])
    num10, num11, den10, den11 = _sc_conv(
        0, xwh1.reshape(2 * N, HID), src_p, dst_rows, aeP,
        as1_flat, ad1_flat, bounds1)

    xwh2, asd2, mx2 = pl.pallas_call(
        _k5_body,
        grid=(N_GRID,),
        in_specs=[pl.BlockSpec((NB, HID), lambda i: (i, 0)),
                  pl.BlockSpec((NB, HID), lambda i: (i, 0)),
                  pl.BlockSpec((NB, 1), lambda i: (i, 0)),
                  pl.BlockSpec((NB, 1), lambda i: (i, 0)),
                  _full_spec((HEADS * HID,)),
                  _full_spec((HEADS * HID, HEADS * OUT)),
                  _full_spec((HEADS * OUT, 2 * HEADS))],
        out_specs=[pl.BlockSpec((2, NB, OUT), lambda i: (0, i, 0)),
                   pl.BlockSpec((NB, 2 * HEADS), lambda i: (i, 0)),
                   pl.BlockSpec((8, 2 * HEADS), lambda i: (0, 0))],
        out_shape=[jax.ShapeDtypeStruct((2, N, OUT), jnp.float32),
                   jax.ShapeDtypeStruct((N, 2 * HEADS), jnp.float32),
                   jax.ShapeDtypeStruct((8, 2 * HEADS), jnp.float32)],
    )(num10, num11, den10.reshape(N, 1), den11.reshape(N, 1),
      bias1, Wc2, att2)

    bounds2 = jnp.concatenate([
        jnp.broadcast_to(mx2[0, 0] + mx2[0, HEADS] + mxe[0, HEADS], (16,)),
        jnp.broadcast_to(mx2[0, 1] + mx2[0, HEADS + 1] + mxe[0, HEADS + 1], (16,))])
    as2_flat = jnp.concatenate([asd2[:, 0], asd2[:, 1]])
    ad2_flat = jnp.concatenate([asd2[:, 2], asd2[:, 3]])
    num20, num21, den20, den21 = _sc_conv(
        2, xwh2.reshape(2 * N, OUT), src_p, dst_rows, aeP,
        as2_flat, ad2_flat, bounds2)

    out = pl.pallas_call(
        _k6_body,
        grid=(N_GRID,),
        in_specs=[pl.BlockSpec((NB, OUT), lambda i: (i, 0)),
                  pl.BlockSpec((NB, OUT), lambda i: (i, 0)),
                  pl.BlockSpec((NB, 1), lambda i: (i, 0)),
                  pl.BlockSpec((NB, 1), lambda i: (i, 0)),
                  _full_spec((HEADS * OUT,))],
        out_specs=pl.BlockSpec((NB, HEADS * OUT), lambda i: (i, 0)),
        out_shape=jax.ShapeDtypeStruct((N, HEADS * OUT), jnp.float32),
    )(num20, num21, den20.reshape(N, 1), den21.reshape(N, 1), bias2)
    return out


# speedup vs baseline: 117.9502x; 1.2084x over previous
import functools

import jax
import jax.numpy as jnp
from jax import lax
from jax.experimental import pallas as pl
from jax.experimental.pallas import tpu as pltpu
from jax.experimental.pallas import tpu_sc as plsc

N = 100000
E = 1600000
D_IN = 128
HID = 16
HEADS = 2
OUT = 16
EDGE_DIM = 8

NB = 2000
N_GRID = N // NB
EB = 12800
E_GRID = E // EB

ECP = 12800
EP = ECP * 128
CR = 4
KE = CR * 128
NCHUNK = ECP // CR
PAIRS = NCHUNK // 32
WB = 400
NWB = N // WB
SWB = 1000
NSWB = N // SWB


def _k1_body(x_ref, w1_ref, b1_ref, t1_ref, s_ref):
    i = pl.program_id(0)
    t1 = jnp.dot(x_ref[...], w1_ref[...], preferred_element_type=jnp.float32)
    t1 = t1 + b1_ref[...][None, :]
    t1_ref[...] = t1

    @pl.when(i == 0)
    def _():
        s_ref[...] = jnp.zeros_like(s_ref)

    s_ref[0:1, :] += jnp.sum(t1, axis=0)[None, :]
    s_ref[1:2, :] += jnp.sum(t1 * t1, axis=0)[None, :]


def _k2_body(t1_ref, s1_ref, w2_ref, b2_ref, g1_ref, be1_ref, t2_ref, s_ref):
    i = pl.program_id(0)
    s1 = s1_ref[...]
    m = s1[0, :] / N
    v = s1[1, :] / N - m * m
    rstd = jax.lax.rsqrt(v + 1e-5)
    h1 = (t1_ref[...] - m[None, :]) * (rstd * g1_ref[...])[None, :] + be1_ref[...][None, :]
    h1 = jnp.maximum(h1, 0.0)
    t2 = jnp.dot(h1, w2_ref[...], preferred_element_type=jnp.float32)
    t2 = t2 + b2_ref[...][None, :]
    t2_ref[...] = t2

    @pl.when(i == 0)
    def _():
        s_ref[...] = jnp.zeros_like(s_ref)

    s_ref[0:1, :] += jnp.sum(t2, axis=0)[None, :]
    s_ref[1:2, :] += jnp.sum(t2 * t2, axis=0)[None, :]


def _k3_body(t2_ref, s2_ref, g_ref, be_ref, wc_ref, att_ref,
             xw_ref, asd_ref, mx_ref):
    i = pl.program_id(0)
    s2 = s2_ref[...]
    m = s2[0, :] / N
    v = s2[1, :] / N - m * m
    rstd = jax.lax.rsqrt(v + 1e-5)
    h = (t2_ref[...] - m[None, :]) * (rstd * g_ref[...])[None, :] + be_ref[...][None, :]
    h = jnp.maximum(h, 0.0)
    xw = jnp.dot(h, wc_ref[...], preferred_element_type=jnp.float32)
    xw_ref[0] = xw[:, :HID]
    xw_ref[1] = xw[:, HID:]
    asd = jnp.dot(xw, att_ref[...], preferred_element_type=jnp.float32)
    asd_ref[...] = asd

    @pl.when(i == 0)
    def _():
        mx_ref[...] = jnp.full_like(mx_ref, -jnp.inf)

    mx_ref[0:1, :] = jnp.maximum(mx_ref[0:1, :], jnp.max(asd, axis=0)[None, :])


def _k5_body(num0_ref, num1_ref, den0_ref, den1_ref, bias_ref, wc_ref,
             att_ref, xw_ref, asd_ref, mx_ref):
    i = pl.program_id(0)
    h0 = num0_ref[...] / (den0_ref[...] + 1e-16)
    h1 = num1_ref[...] / (den1_ref[...] + 1e-16)
    h = jnp.concatenate([h0, h1], axis=1) + bias_ref[...][None, :]
    h = jnp.maximum(h, 0.0)
    xw = jnp.dot(h, wc_ref[...], preferred_element_type=jnp.float32)
    xw_ref[0] = xw[:, :OUT]
    xw_ref[1] = xw[:, OUT:]
    asd = jnp.dot(xw, att_ref[...], preferred_element_type=jnp.float32)
    asd_ref[...] = asd

    @pl.when(i == 0)
    def _():
        mx_ref[...] = jnp.full_like(mx_ref, -jnp.inf)

    mx_ref[0:1, :] = jnp.maximum(mx_ref[0:1, :], jnp.max(asd, axis=0)[None, :])


def _k4_body(ea_ref, ae_matt_ref, aeo_ref, mx_ref):
    i = pl.program_id(0)
    aeo = jax.lax.dot_general(ae_matt_ref[...], ea_ref[...],
                              (((1,), (1,)), ((), ())),
                              preferred_element_type=jnp.float32)

    @pl.when(i == 0)
    def _():
        mx_ref[...] = jnp.full_like(mx_ref, -jnp.inf)

    @pl.when(i < E_GRID)
    def _():
        aeo_ref[...] = aeo
        mx_ref[0:1, :] = jnp.maximum(mx_ref[0:1, :],
                                     jnp.max(aeo, axis=1)[None, :])

    @pl.when(i >= E_GRID)
    def _():
        aeo_ref[...] = jnp.full_like(aeo_ref, -jnp.inf)


def _k6_body(num0_ref, num1_ref, den0_ref, den1_ref, bias_ref, out_ref):
    h0 = num0_ref[...] / (den0_ref[...] + 1e-16)
    h1 = num1_ref[...] / (den1_ref[...] + 1e-16)
    out_ref[...] = jnp.concatenate([h0, h1], axis=1) + bias_ref[...][None, :]


def _full_spec(shape):
    return pl.BlockSpec(shape, lambda i: tuple(0 for _ in shape))


def _node_pipeline(x, W1, b1, g1, be1, W2, b2, g_emb, be_emb, Wc1, att1):
    t1, s1 = pl.pallas_call(
        _k1_body,
        grid=(N_GRID,),
        in_specs=[pl.BlockSpec((NB, D_IN), lambda i: (i, 0)),
                  _full_spec((D_IN, HID)), _full_spec((HID,))],
        out_specs=[pl.BlockSpec((NB, HID), lambda i: (i, 0)),
                   pl.BlockSpec((8, HID), lambda i: (0, 0))],
        out_shape=[jax.ShapeDtypeStruct((N, HID), jnp.float32),
                   jax.ShapeDtypeStruct((8, HID), jnp.float32)],
    )(x, W1, b1)
    t2, s2 = pl.pallas_call(
        _k2_body,
        grid=(N_GRID,),
        in_specs=[pl.BlockSpec((NB, HID), lambda i: (i, 0)),
                  _full_spec((8, HID)), _full_spec((HID, HID)),
                  _full_spec((HID,)), _full_spec((HID,)), _full_spec((HID,))],
        out_specs=[pl.BlockSpec((NB, HID), lambda i: (i, 0)),
                   pl.BlockSpec((8, HID), lambda i: (0, 0))],
        out_shape=[jax.ShapeDtypeStruct((N, HID), jnp.float32),
                   jax.ShapeDtypeStruct((8, HID), jnp.float32)],
    )(t1, s1, W2, b2, g1, be1)
    xwh, asd, mx = pl.pallas_call(
        _k3_body,
        grid=(N_GRID,),
        in_specs=[pl.BlockSpec((NB, HID), lambda i: (i, 0)),
                  _full_spec((8, HID)), _full_spec((HID,)), _full_spec((HID,)),
                  _full_spec((HID, 2 * HID)), _full_spec((2 * HID, 2 * HEADS))],
        out_specs=[pl.BlockSpec((2, NB, HID), lambda i: (0, i, 0)),
                   pl.BlockSpec((NB, 2 * HEADS), lambda i: (i, 0)),
                   pl.BlockSpec((8, 2 * HEADS), lambda i: (0, 0))],
        out_shape=[jax.ShapeDtypeStruct((2, N, HID), jnp.float32),
                   jax.ShapeDtypeStruct((N, 2 * HEADS), jnp.float32),
                   jax.ShapeDtypeStruct((8, 2 * HEADS), jnp.float32)],
    )(t2, s2, g_emb, be_emb, Wc1, att1)
    return xwh, asd, mx


def _att_mat(att_s, att_d, H, C):
    m = jnp.zeros((H * C, 2 * H), jnp.float32)
    for h in range(H):
        m = m.at[h * C:(h + 1) * C, h].set(att_s[h])
        m = m.at[h * C:(h + 1) * C, H + h].set(att_d[h])
    return m


def _sc_conv_body(base, xw_hbm, src_hbm, dstr_hbm, ae_hbm, bnd_hbm,
                  as_hbm, ad_hbm,
                  num0_out, num1_out, den0_out, den1_out,
                  num_sp, den_sp,
                  srcbA, dstbA, dstfbA, aebA, asbA, adbA, exbA, msgA,
                  srcbB, dstbB, dstfbB, aebB, asbB, adbB, exbB, msgB, bnd,
                  sem_inA, sem_gA, sem_sA, sem_inB, sem_gB, sem_sB):
    c = lax.axis_index("c")
    s = lax.axis_index("s")

    pltpu.sync_copy(bnd_hbm.at[pl.ds(c * 16, 16)], bnd)

    @pl.loop(0, KE)
    def _(r):
        msgA[r, :] = jnp.zeros((HID,), jnp.float32)

    @pl.loop(0, (KE + 16) // 16)
    def _(r):
        exbA[pl.ds(r * 16, 16)] = jnp.zeros((16,), jnp.float32)

    @pl.loop(s, NWB, step=16)
    def _(i):
        sl = pl.ds(i * WB, WB)
        pltpu.sync_copy(msgA.at[pl.ds(0, WB)], num_sp.at[sl])
        pltpu.sync_copy(exbA.at[pl.ds(0, WB)], den_sp.at[sl])

    plsc.subcore_barrier()

    bndv = bnd[...]
    cn = c * N

    def issue_in(bs, i):
        e0 = i * KE
        pltpu.async_copy(src_hbm.at[pl.ds(e0, KE)], bs[0], bs[8])
        pltpu.async_copy(dstr_hbm.at[pl.ds(i * CR, CR)], bs[1], bs[8])
        pltpu.async_copy(ae_hbm.at[base + c, pl.ds(e0, KE)], bs[3], bs[8])

    def wait_in(bs, i):
        e0 = i * KE
        pltpu.make_async_copy(src_hbm.at[pl.ds(e0, KE)], bs[0], bs[8]).wait()
        pltpu.make_async_copy(dstr_hbm.at[pl.ds(i * CR, CR)], bs[1],
                              bs[8]).wait()
        pltpu.make_async_copy(ae_hbm.at[base + c, pl.ds(e0, KE)], bs[3],
                              bs[8]).wait()

    def build_idx(bs):
        srcb, dstb, dstfb = bs[0], bs[1], bs[2]

        @pl.loop(0, KE // 16)
        def _(t):
            sl = pl.ds(t * 16, 16)
            srcb[sl] = srcb[sl] + cn
        for j in range(CR):
            for k in range(8):
                dstfb[pl.ds((j * 8 + k) * 16, 16)] = (
                    dstb[j, pl.ds(k * 16, 16)] + cn)

    def issue_gathers(bs):
        srcb, dstfb, asb, adb, msg, semg = bs[0], bs[2], bs[4], bs[5], bs[7], bs[9]
        g = [pltpu.async_copy(as_hbm.at[srcb.at[pl.ds(j * 128, 128)]],
                              asb.at[pl.ds(j * 128, 128)], semg)
             for j in range(CR)]
        g += [pltpu.async_copy(ad_hbm.at[dstfb.at[pl.ds(j * 128, 128)]],
                               adb.at[pl.ds(j * 128, 128)], semg)
              for j in range(CR)]
        g += [pltpu.async_copy(xw_hbm.at[srcb.at[pl.ds(j * 128, 128)]],
                               msg.at[pl.ds(j * 128, 128)], semg)
              for j in range(CR)]
        return g

    def compute(bs):
        aeb, asb, adb, exb, msg = bs[3], bs[4], bs[5], bs[6], bs[7]

        @pl.loop(0, KE // 16)
        def _(t):
            sl = pl.ds(t * 16, 16)
            a = asb[sl] + adb[sl] + aeb[sl]
            lr = jnp.maximum(a, 0.2 * a)
            exb[sl] = jnp.exp(lr - bndv)

        @pl.loop(0, KE // 16)
        def _(g_):
            exv = exb[pl.ds(g_ * 16, 16)]
            b0 = g_ * 16
            for k in range(16):
                msg[b0 + k, :] = msg[b0 + k, :] * exv[k]

    def issue_scatters(bs):
        dstb, exb, msg, sems = bs[1], bs[6], bs[7], bs[10]
        sc = []
        for j in range(CR):
            sc.append(pltpu.async_copy(msg.at[pl.ds(j * 128, 128)],
                                       num_sp.at[dstb.at[j]], sems, add=True))
            sc.append(pltpu.async_copy(exb.at[pl.ds(j * 128, 128)],
                                       den_sp.at[dstb.at[j]], sems, add=True))
        return sc

    bsA = (srcbA, dstbA, dstfbA, aebA, asbA, adbA, exbA, msgA,
           sem_inA, sem_gA, sem_sA)
    bsB = (srcbB, dstbB, dstfbB, aebB, asbB, adbB, exbB, msgB,
           sem_inB, sem_gB, sem_sB)

    issue_in(bsA, s)
    issue_in(bsB, s + 16)

    @pl.loop(0, PAIRS)
    def _(p):
        iA = s + 32 * p
        iB = iA + 16
        wait_in(bsA, iA)
        build_idx(bsA)
        gA = issue_gathers(bsA)
        wait_in(bsB, iB)
        build_idx(bsB)
        gB = issue_gathers(bsB)
        for cp in gA:
            cp.wait()
        compute(bsA)
        scA = issue_scatters(bsA)
        for cp in gB:
            cp.wait()
        compute(bsB)
        scB = issue_scatters(bsB)
        for cp in scA:
            cp.wait()

        @pl.when(p < PAIRS - 1)
        def _():
            issue_in(bsA, iA + 32)

        for cp in scB:
            cp.wait()

        @pl.when(p < PAIRS - 1)
        def _():
            issue_in(bsB, iB + 32)

    plsc.subcore_barrier()

    @pl.loop(s, NWB, step=16)
    def _(i):
        sl = pl.ds(i * WB, WB)
        pltpu.sync_copy(num_sp.at[sl], msgA.at[pl.ds(0, WB)])
        pltpu.sync_copy(den_sp.at[sl], exbA.at[pl.ds(0, WB)])

        @pl.when(c == 0)
        def _():
            pltpu.sync_copy(msgA.at[pl.ds(0, WB)], num0_out.at[sl])
            pltpu.sync_copy(exbA.at[pl.ds(0, WB)], den0_out.at[sl])

        @pl.when(c == 1)
        def _():
            pltpu.sync_copy(msgA.at[pl.ds(0, WB)], num1_out.at[sl])
            pltpu.sync_copy(exbA.at[pl.ds(0, WB)], den1_out.at[sl])


def _sc_conv(base, xwflat, src_flat, dst_rows, aeP, as_flat, ad_flat, bounds):
    mesh = plsc.VectorSubcoreMesh(core_axis_name="c", subcore_axis_name="s")
    return pl.kernel(
        functools.partial(_sc_conv_body, base),
        out_type=[jax.ShapeDtypeStruct((N, HID), jnp.float32),
                  jax.ShapeDtypeStruct((N, HID), jnp.float32),
                  jax.ShapeDtypeStruct((N,), jnp.float32),
                  jax.ShapeDtypeStruct((N,), jnp.float32)],
        mesh=mesh,
        compiler_params=pltpu.CompilerParams(use_tc_tiling_on_sc=False),
        scratch_types=(
            [pltpu.VMEM_SHARED((N, HID), jnp.float32),
             pltpu.VMEM_SHARED((N,), jnp.float32)]
            + 2 * [pltpu.VMEM((KE,), jnp.int32),
                   pltpu.VMEM((CR, 128), jnp.int32),
                   pltpu.VMEM((KE,), jnp.int32),
                   pltpu.VMEM((KE,), jnp.float32),
                   pltpu.VMEM((KE,), jnp.float32),
                   pltpu.VMEM((KE,), jnp.float32),
                   pltpu.VMEM((KE + 16,), jnp.float32),
                   pltpu.VMEM((KE, HID), jnp.float32)]
            + [pltpu.VMEM((16,), jnp.float32)]
            + 6 * [pltpu.SemaphoreType.DMA]
        ),
    )(xwflat, src_flat, dst_rows, aeP, bounds, as_flat, ad_flat)


def kernel(x, edge_index, edge_attr, W1, b1, g1, be1, W2, b2, g_emb, be_emb,
           Wc1, as1, ad1, We1, ae1, bias1, Wc2, as2, ad2, We2, ae2, bias2):
    src = edge_index[0].astype(jnp.int32)
    dst = edge_index[1].astype(jnp.int32)
    src_p = jnp.concatenate([src, jnp.zeros((EP - E,), jnp.int32)])
    dst_p = jnp.concatenate([dst, jnp.zeros((EP - E,), jnp.int32)])
    dst_rows = dst_p.reshape(ECP, 128)

    att1 = _att_mat(as1, ad1, HEADS, HID)
    att2 = _att_mat(as2, ad2, HEADS, OUT)
    Ae1 = (We1.reshape(EDGE_DIM, HEADS, HID) * ae1[None, :, :]).sum(-1)
    Ae2 = (We2.reshape(EDGE_DIM, HEADS, OUT) * ae2[None, :, :]).sum(-1)
    AeAll = jnp.concatenate([Ae1, Ae2], axis=1)

    xwh1, asd1, mx1 = _node_pipeline(x, W1, b1, g1, be1, W2, b2,
                                     g_emb, be_emb, Wc1, att1)

    aeP, mxe = pl.pallas_call(
        _k4_body,
        grid=(EP // EB,),
        in_specs=[pl.BlockSpec((EB, EDGE_DIM),
                               lambda i: (jnp.minimum(i, E_GRID - 1), 0)),
                  _full_spec((2 * HEADS, EDGE_DIM))],
        out_specs=[pl.BlockSpec((2 * HEADS, EB), lambda i: (0, i)),
                   pl.BlockSpec((8, 2 * HEADS), lambda i: (0, 0))],
        out_shape=[jax.ShapeDtypeStruct((2 * HEADS, EP), jnp.float32),
                   jax.ShapeDtypeStruct((8, 2 * HEADS), jnp.float32)],
    )(edge_attr, AeAll.T)

    bounds1 = jnp.concatenate([
        jnp.broadcast_to(mx1[0, 0] + mx1[0, HEADS] + mxe[0, 0], (16,)),
        jnp.broadcast_to(mx1[0, 1] + mx1[0, HEADS + 1] + mxe[0, 1], (16,))])
    as1_flat = jnp.concatenate([asd1[:, 0], asd1[:, 1]])
    ad1_flat = jnp.concatenate([asd1[:, 2], asd1[:, 3]])
    num10, num11, den10, den11 = _sc_conv(
        0, xwh1.reshape(2 * N, HID), src_p, dst_rows, aeP,
        as1_flat, ad1_flat, bounds1)

    xwh2, asd2, mx2 = pl.pallas_call(
        _k5_body,
        grid=(N_GRID,),
        in_specs=[pl.BlockSpec((NB, HID), lambda i: (i, 0)),
                  pl.BlockSpec((NB, HID), lambda i: (i, 0)),
                  pl.BlockSpec((NB, 1), lambda i: (i, 0)),
                  pl.BlockSpec((NB, 1), lambda i: (i, 0)),
                  _full_spec((HEADS * HID,)),
                  _full_spec((HEADS * HID, HEADS * OUT)),
                  _full_spec((HEADS * OUT, 2 * HEADS))],
        out_specs=[pl.BlockSpec((2, NB, OUT), lambda i: (0, i, 0)),
                   pl.BlockSpec((NB, 2 * HEADS), lambda i: (i, 0)),
                   pl.BlockSpec((8, 2 * HEADS), lambda i: (0, 0))],
        out_shape=[jax.ShapeDtypeStruct((2, N, OUT), jnp.float32),
                   jax.ShapeDtypeStruct((N, 2 * HEADS), jnp.float32),
                   jax.ShapeDtypeStruct((8, 2 * HEADS), jnp.float32)],
    )(num10, num11, den10.reshape(N, 1), den11.reshape(N, 1),
      bias1, Wc2, att2)

    bounds2 = jnp.concatenate([
        jnp.broadcast_to(mx2[0, 0] + mx2[0, HEADS] + mxe[0, HEADS], (16,)),
        jnp.broadcast_to(mx2[0, 1] + mx2[0, HEADS + 1] + mxe[0, HEADS + 1], (16,))])
    as2_flat = jnp.concatenate([asd2[:, 0], asd2[:, 1]])
    ad2_flat = jnp.concatenate([asd2[:, 2], asd2[:, 3]])
    num20, num21, den20, den21 = _sc_conv(
        2, xwh2.reshape(2 * N, OUT), src_p, dst_rows, aeP,
        as2_flat, ad2_flat, bounds2)

    out = pl.pallas_call(
        _k6_body,
        grid=(N_GRID,),
        in_specs=[pl.BlockSpec((NB, OUT), lambda i: (i, 0)),
                  pl.BlockSpec((NB, OUT), lambda i: (i, 0)),
                  pl.BlockSpec((NB, 1), lambda i: (i, 0)),
                  pl.BlockSpec((NB, 1), lambda i: (i, 0)),
                  _full_spec((HEADS * OUT,))],
        out_specs=pl.BlockSpec((NB, HEADS * OUT), lambda i: (i, 0)),
        out_shape=jax.ShapeDtypeStruct((N, HEADS * OUT), jnp.float32),
    )(num20, num21, den20.reshape(N, 1), den21.reshape(N, 1), bias2)
    return out


# speedup vs baseline: 120.3832x; 1.0206x over previous
import functools

import jax
import jax.numpy as jnp
from jax import lax
from jax.experimental import pallas as pl
from jax.experimental.pallas import tpu as pltpu
from jax.experimental.pallas import tpu_sc as plsc

N = 100000
E = 1600000
D_IN = 128
HID = 16
HEADS = 2
OUT = 16
EDGE_DIM = 8

NB = 5000
N_GRID = N // NB
EB = 12800
E_GRID = E // EB

ECP = 12800
EP = ECP * 128
CR = 4
KE = CR * 128
NCHUNK = ECP // CR
PAIRS = NCHUNK // 32
WB = 400
NWB = N // WB
SWB = 1000
NSWB = N // SWB


def _k1_body(x_ref, w1_ref, b1_ref, t1_ref, s_ref):
    i = pl.program_id(0)
    t1 = jnp.dot(x_ref[...], w1_ref[...], preferred_element_type=jnp.float32)
    t1 = t1 + b1_ref[...][None, :]
    t1_ref[...] = t1

    @pl.when(i == 0)
    def _():
        s_ref[...] = jnp.zeros_like(s_ref)

    s_ref[0:1, :] += jnp.sum(t1, axis=0)[None, :]
    s_ref[1:2, :] += jnp.sum(t1 * t1, axis=0)[None, :]


def _k2_body(t1_ref, s1_ref, w2_ref, b2_ref, g1_ref, be1_ref, t2_ref, s_ref):
    i = pl.program_id(0)
    s1 = s1_ref[...]
    m = s1[0, :] / N
    v = s1[1, :] / N - m * m
    rstd = jax.lax.rsqrt(v + 1e-5)
    h1 = (t1_ref[...] - m[None, :]) * (rstd * g1_ref[...])[None, :] + be1_ref[...][None, :]
    h1 = jnp.maximum(h1, 0.0)
    t2 = jnp.dot(h1, w2_ref[...], preferred_element_type=jnp.float32)
    t2 = t2 + b2_ref[...][None, :]
    t2_ref[...] = t2

    @pl.when(i == 0)
    def _():
        s_ref[...] = jnp.zeros_like(s_ref)

    s_ref[0:1, :] += jnp.sum(t2, axis=0)[None, :]
    s_ref[1:2, :] += jnp.sum(t2 * t2, axis=0)[None, :]


def _k3_body(t2_ref, s2_ref, g_ref, be_ref, wc_ref, att_ref,
             xw_ref, asd_ref, mx_ref):
    i = pl.program_id(0)
    s2 = s2_ref[...]
    m = s2[0, :] / N
    v = s2[1, :] / N - m * m
    rstd = jax.lax.rsqrt(v + 1e-5)
    h = (t2_ref[...] - m[None, :]) * (rstd * g_ref[...])[None, :] + be_ref[...][None, :]
    h = jnp.maximum(h, 0.0)
    xw = jnp.dot(h, wc_ref[...], preferred_element_type=jnp.float32)
    xw_ref[0] = xw[:, :HID]
    xw_ref[1] = xw[:, HID:]
    asd = jnp.dot(xw, att_ref[...], preferred_element_type=jnp.float32)
    asd_ref[...] = asd

    @pl.when(i == 0)
    def _():
        mx_ref[...] = jnp.full_like(mx_ref, -jnp.inf)

    mx_ref[0:1, :] = jnp.maximum(mx_ref[0:1, :], jnp.max(asd, axis=0)[None, :])


def _k5_body(num0_ref, num1_ref, den0_ref, den1_ref, bias_ref, wc_ref,
             att_ref, xw_ref, asd_ref, mx_ref):
    i = pl.program_id(0)
    h0 = num0_ref[...] / (den0_ref[...] + 1e-16)
    h1 = num1_ref[...] / (den1_ref[...] + 1e-16)
    h = jnp.concatenate([h0, h1], axis=1) + bias_ref[...][None, :]
    h = jnp.maximum(h, 0.0)
    xw = jnp.dot(h, wc_ref[...], preferred_element_type=jnp.float32)
    xw_ref[0] = xw[:, :OUT]
    xw_ref[1] = xw[:, OUT:]
    asd = jnp.dot(xw, att_ref[...], preferred_element_type=jnp.float32)
    asd_ref[...] = asd

    @pl.when(i == 0)
    def _():
        mx_ref[...] = jnp.full_like(mx_ref, -jnp.inf)

    mx_ref[0:1, :] = jnp.maximum(mx_ref[0:1, :], jnp.max(asd, axis=0)[None, :])


def _k4_body(ea_ref, ae_matt_ref, aeo_ref, mx_ref):
    i = pl.program_id(0)
    aeo = jax.lax.dot_general(ae_matt_ref[...], ea_ref[...],
                              (((1,), (1,)), ((), ())),
                              preferred_element_type=jnp.float32)

    @pl.when(i == 0)
    def _():
        mx_ref[...] = jnp.full_like(mx_ref, -jnp.inf)

    @pl.when(i < E_GRID)
    def _():
        aeo_ref[...] = aeo
        mx_ref[0:1, :] = jnp.maximum(mx_ref[0:1, :],
                                     jnp.max(aeo, axis=1)[None, :])

    @pl.when(i >= E_GRID)
    def _():
        aeo_ref[...] = jnp.full_like(aeo_ref, -jnp.inf)


def _k6_body(num0_ref, num1_ref, den0_ref, den1_ref, bias_ref, out_ref):
    h0 = num0_ref[...] / (den0_ref[...] + 1e-16)
    h1 = num1_ref[...] / (den1_ref[...] + 1e-16)
    out_ref[...] = jnp.concatenate([h0, h1], axis=1) + bias_ref[...][None, :]


def _full_spec(shape):
    return pl.BlockSpec(shape, lambda i: tuple(0 for _ in shape))


def _node_pipeline(x, W1, b1, g1, be1, W2, b2, g_emb, be_emb, Wc1, att1):
    t1, s1 = pl.pallas_call(
        _k1_body,
        grid=(N_GRID,),
        in_specs=[pl.BlockSpec((NB, D_IN), lambda i: (i, 0)),
                  _full_spec((D_IN, HID)), _full_spec((HID,))],
        out_specs=[pl.BlockSpec((NB, HID), lambda i: (i, 0)),
                   pl.BlockSpec((8, HID), lambda i: (0, 0))],
        out_shape=[jax.ShapeDtypeStruct((N, HID), jnp.float32),
                   jax.ShapeDtypeStruct((8, HID), jnp.float32)],
    )(x, W1, b1)
    t2, s2 = pl.pallas_call(
        _k2_body,
        grid=(N_GRID,),
        in_specs=[pl.BlockSpec((NB, HID), lambda i: (i, 0)),
                  _full_spec((8, HID)), _full_spec((HID, HID)),
                  _full_spec((HID,)), _full_spec((HID,)), _full_spec((HID,))],
        out_specs=[pl.BlockSpec((NB, HID), lambda i: (i, 0)),
                   pl.BlockSpec((8, HID), lambda i: (0, 0))],
        out_shape=[jax.ShapeDtypeStruct((N, HID), jnp.float32),
                   jax.ShapeDtypeStruct((8, HID), jnp.float32)],
    )(t1, s1, W2, b2, g1, be1)
    xwh, asd, mx = pl.pallas_call(
        _k3_body,
        grid=(N_GRID,),
        in_specs=[pl.BlockSpec((NB, HID), lambda i: (i, 0)),
                  _full_spec((8, HID)), _full_spec((HID,)), _full_spec((HID,)),
                  _full_spec((HID, 2 * HID)), _full_spec((2 * HID, 2 * HEADS))],
        out_specs=[pl.BlockSpec((2, NB, HID), lambda i: (0, i, 0)),
                   pl.BlockSpec((NB, 2 * HEADS), lambda i: (i, 0)),
                   pl.BlockSpec((8, 2 * HEADS), lambda i: (0, 0))],
        out_shape=[jax.ShapeDtypeStruct((2, N, HID), jnp.float32),
                   jax.ShapeDtypeStruct((N, 2 * HEADS), jnp.float32),
                   jax.ShapeDtypeStruct((8, 2 * HEADS), jnp.float32)],
    )(t2, s2, g_emb, be_emb, Wc1, att1)
    return xwh, asd, mx


def _att_mat(att_s, att_d, H, C):
    m = jnp.zeros((H * C, 2 * H), jnp.float32)
    for h in range(H):
        m = m.at[h * C:(h + 1) * C, h].set(att_s[h])
        m = m.at[h * C:(h + 1) * C, H + h].set(att_d[h])
    return m


def _sc_conv_body(base, xw_hbm, src_hbm, dstr_hbm, ae_hbm, bnd_hbm,
                  as_hbm, ad_hbm,
                  num0_out, num1_out, den0_out, den1_out,
                  num_sp, den_sp,
                  srcbA, dstbA, dstfbA, aebA, asbA, adbA, exbA, msgA,
                  srcbB, dstbB, dstfbB, aebB, asbB, adbB, exbB, msgB, bnd,
                  sem_inA, sem_gA, sem_sA, sem_inB, sem_gB, sem_sB):
    c = lax.axis_index("c")
    s = lax.axis_index("s")

    pltpu.sync_copy(bnd_hbm.at[pl.ds(c * 16, 16)], bnd)

    @pl.loop(0, KE)
    def _(r):
        msgA[r, :] = jnp.zeros((HID,), jnp.float32)

    @pl.loop(0, (KE + 16) // 16)
    def _(r):
        exbA[pl.ds(r * 16, 16)] = jnp.zeros((16,), jnp.float32)

    @pl.loop(s, NWB, step=16)
    def _(i):
        sl = pl.ds(i * WB, WB)
        pltpu.sync_copy(msgA.at[pl.ds(0, WB)], num_sp.at[sl])
        pltpu.sync_copy(exbA.at[pl.ds(0, WB)], den_sp.at[sl])

    plsc.subcore_barrier()

    bndv = bnd[...]
    cn = c * N

    def issue_in(bs, i):
        e0 = i * KE
        pltpu.async_copy(src_hbm.at[pl.ds(e0, KE)], bs[0], bs[8])
        pltpu.async_copy(dstr_hbm.at[pl.ds(i * CR, CR)], bs[1], bs[8])
        pltpu.async_copy(ae_hbm.at[base + c, pl.ds(e0, KE)], bs[3], bs[8])

    def wait_in(bs, i):
        e0 = i * KE
        pltpu.make_async_copy(src_hbm.at[pl.ds(e0, KE)], bs[0], bs[8]).wait()
        pltpu.make_async_copy(dstr_hbm.at[pl.ds(i * CR, CR)], bs[1],
                              bs[8]).wait()
        pltpu.make_async_copy(ae_hbm.at[base + c, pl.ds(e0, KE)], bs[3],
                              bs[8]).wait()

    def build_idx(bs):
        srcb, dstb, dstfb = bs[0], bs[1], bs[2]

        @pl.loop(0, KE // 16)
        def _(t):
            sl = pl.ds(t * 16, 16)
            srcb[sl] = srcb[sl] + cn
        for j in range(CR):
            for k in range(8):
                dstfb[pl.ds((j * 8 + k) * 16, 16)] = (
                    dstb[j, pl.ds(k * 16, 16)] + cn)

    def issue_gathers(bs):
        srcb, dstfb, asb, adb, msg, semg = bs[0], bs[2], bs[4], bs[5], bs[7], bs[9]
        g = [pltpu.async_copy(as_hbm.at[srcb.at[pl.ds(j * 128, 128)]],
                              asb.at[pl.ds(j * 128, 128)], semg)
             for j in range(CR)]
        g += [pltpu.async_copy(ad_hbm.at[dstfb.at[pl.ds(j * 128, 128)]],
                               adb.at[pl.ds(j * 128, 128)], semg)
              for j in range(CR)]
        g += [pltpu.async_copy(xw_hbm.at[srcb.at[pl.ds(j * 128, 128)]],
                               msg.at[pl.ds(j * 128, 128)], semg)
              for j in range(CR)]
        return g

    def compute(bs):
        aeb, asb, adb, exb, msg = bs[3], bs[4], bs[5], bs[6], bs[7]

        @pl.loop(0, KE // 16)
        def _(t):
            sl = pl.ds(t * 16, 16)
            a = asb[sl] + adb[sl] + aeb[sl]
            lr = jnp.maximum(a, 0.2 * a)
            exb[sl] = jnp.exp(lr - bndv)

        @pl.loop(0, KE // 16)
        def _(g_):
            exv = exb[pl.ds(g_ * 16, 16)]
            b0 = g_ * 16
            for k in range(16):
                msg[b0 + k, :] = msg[b0 + k, :] * exv[k]

    def issue_scatters(bs):
        dstb, exb, msg, sems = bs[1], bs[6], bs[7], bs[10]
        sc = []
        for j in range(CR):
            sc.append(pltpu.async_copy(msg.at[pl.ds(j * 128, 128)],
                                       num_sp.at[dstb.at[j]], sems, add=True))
            sc.append(pltpu.async_copy(exb.at[pl.ds(j * 128, 128)],
                                       den_sp.at[dstb.at[j]], sems, add=True))
        return sc

    bsA = (srcbA, dstbA, dstfbA, aebA, asbA, adbA, exbA, msgA,
           sem_inA, sem_gA, sem_sA)
    bsB = (srcbB, dstbB, dstfbB, aebB, asbB, adbB, exbB, msgB,
           sem_inB, sem_gB, sem_sB)

    issue_in(bsA, s)
    issue_in(bsB, s + 16)

    @pl.loop(0, PAIRS)
    def _(p):
        iA = s + 32 * p
        iB = iA + 16
        wait_in(bsA, iA)
        build_idx(bsA)
        gA = issue_gathers(bsA)
        wait_in(bsB, iB)
        build_idx(bsB)
        gB = issue_gathers(bsB)
        for cp in gA:
            cp.wait()
        compute(bsA)
        scA = issue_scatters(bsA)
        for cp in gB:
            cp.wait()
        compute(bsB)
        scB = issue_scatters(bsB)
        for cp in scA:
            cp.wait()

        @pl.when(p < PAIRS - 1)
        def _():
            issue_in(bsA, iA + 32)

        for cp in scB:
            cp.wait()

        @pl.when(p < PAIRS - 1)
        def _():
            issue_in(bsB, iB + 32)

    plsc.subcore_barrier()

    @pl.loop(s, NWB, step=16)
    def _(i):
        sl = pl.ds(i * WB, WB)
        pltpu.sync_copy(num_sp.at[sl], msgA.at[pl.ds(0, WB)])
        pltpu.sync_copy(den_sp.at[sl], exbA.at[pl.ds(0, WB)])

        @pl.when(c == 0)
        def _():
            pltpu.sync_copy(msgA.at[pl.ds(0, WB)], num0_out.at[sl])
            pltpu.sync_copy(exbA.at[pl.ds(0, WB)], den0_out.at[sl])

        @pl.when(c == 1)
        def _():
            pltpu.sync_copy(msgA.at[pl.ds(0, WB)], num1_out.at[sl])
            pltpu.sync_copy(exbA.at[pl.ds(0, WB)], den1_out.at[sl])


def _sc_conv(base, xwflat, src_flat, dst_rows, aeP, as_flat, ad_flat, bounds):
    mesh = plsc.VectorSubcoreMesh(core_axis_name="c", subcore_axis_name="s")
    return pl.kernel(
        functools.partial(_sc_conv_body, base),
        out_type=[jax.ShapeDtypeStruct((N, HID), jnp.float32),
                  jax.ShapeDtypeStruct((N, HID), jnp.float32),
                  jax.ShapeDtypeStruct((N,), jnp.float32),
                  jax.ShapeDtypeStruct((N,), jnp.float32)],
        mesh=mesh,
        compiler_params=pltpu.CompilerParams(use_tc_tiling_on_sc=False),
        scratch_types=(
            [pltpu.VMEM_SHARED((N, HID), jnp.float32),
             pltpu.VMEM_SHARED((N,), jnp.float32)]
            + 2 * [pltpu.VMEM((KE,), jnp.int32),
                   pltpu.VMEM((CR, 128), jnp.int32),
                   pltpu.VMEM((KE,), jnp.int32),
                   pltpu.VMEM((KE,), jnp.float32),
                   pltpu.VMEM((KE,), jnp.float32),
                   pltpu.VMEM((KE,), jnp.float32),
                   pltpu.VMEM((KE + 16,), jnp.float32),
                   pltpu.VMEM((KE, HID), jnp.float32)]
            + [pltpu.VMEM((16,), jnp.float32)]
            + 6 * [pltpu.SemaphoreType.DMA]
        ),
    )(xwflat, src_flat, dst_rows, aeP, bounds, as_flat, ad_flat)


def kernel(x, edge_index, edge_attr, W1, b1, g1, be1, W2, b2, g_emb, be_emb,
           Wc1, as1, ad1, We1, ae1, bias1, Wc2, as2, ad2, We2, ae2, bias2):
    src = edge_index[0].astype(jnp.int32)
    dst = edge_index[1].astype(jnp.int32)
    src_p = jnp.concatenate([src, jnp.zeros((EP - E,), jnp.int32)])
    dst_p = jnp.concatenate([dst, jnp.zeros((EP - E,), jnp.int32)])
    dst_rows = dst_p.reshape(ECP, 128)

    att1 = _att_mat(as1, ad1, HEADS, HID)
    att2 = _att_mat(as2, ad2, HEADS, OUT)
    Ae1 = (We1.reshape(EDGE_DIM, HEADS, HID) * ae1[None, :, :]).sum(-1)
    Ae2 = (We2.reshape(EDGE_DIM, HEADS, OUT) * ae2[None, :, :]).sum(-1)
    AeAll = jnp.concatenate([Ae1, Ae2], axis=1)

    xwh1, asd1, mx1 = _node_pipeline(x, W1, b1, g1, be1, W2, b2,
                                     g_emb, be_emb, Wc1, att1)

    aeP, mxe = pl.pallas_call(
        _k4_body,
        grid=(EP // EB,),
        in_specs=[pl.BlockSpec((EB, EDGE_DIM),
                               lambda i: (jnp.minimum(i, E_GRID - 1), 0)),
                  _full_spec((2 * HEADS, EDGE_DIM))],
        out_specs=[pl.BlockSpec((2 * HEADS, EB), lambda i: (0, i)),
                   pl.BlockSpec((8, 2 * HEADS), lambda i: (0, 0))],
        out_shape=[jax.ShapeDtypeStruct((2 * HEADS, EP), jnp.float32),
                   jax.ShapeDtypeStruct((8, 2 * HEADS), jnp.float32)],
    )(edge_attr, AeAll.T)

    bounds1 = jnp.concatenate([
        jnp.broadcast_to(mx1[0, 0] + mx1[0, HEADS] + mxe[0, 0], (16,)),
        jnp.broadcast_to(mx1[0, 1] + mx1[0, HEADS + 1] + mxe[0, 1], (16,))])
    as1_flat = jnp.concatenate([asd1[:, 0], asd1[:, 1]])
    ad1_flat = jnp.concatenate([asd1[:, 2], asd1[:, 3]])
    num10, num11, den10, den11 = _sc_conv(
        0, xwh1.reshape(2 * N, HID), src_p, dst_rows, aeP,
        as1_flat, ad1_flat, bounds1)

    xwh2, asd2, mx2 = pl.pallas_call(
        _k5_body,
        grid=(N_GRID,),
        in_specs=[pl.BlockSpec((NB, HID), lambda i: (i, 0)),
                  pl.BlockSpec((NB, HID), lambda i: (i, 0)),
                  pl.BlockSpec((NB, 1), lambda i: (i, 0)),
                  pl.BlockSpec((NB, 1), lambda i: (i, 0)),
                  _full_spec((HEADS * HID,)),
                  _full_spec((HEADS * HID, HEADS * OUT)),
                  _full_spec((HEADS * OUT, 2 * HEADS))],
        out_specs=[pl.BlockSpec((2, NB, OUT), lambda i: (0, i, 0)),
                   pl.BlockSpec((NB, 2 * HEADS), lambda i: (i, 0)),
                   pl.BlockSpec((8, 2 * HEADS), lambda i: (0, 0))],
        out_shape=[jax.ShapeDtypeStruct((2, N, OUT), jnp.float32),
                   jax.ShapeDtypeStruct((N, 2 * HEADS), jnp.float32),
                   jax.ShapeDtypeStruct((8, 2 * HEADS), jnp.float32)],
    )(num10, num11, den10.reshape(N, 1), den11.reshape(N, 1),
      bias1, Wc2, att2)

    bounds2 = jnp.concatenate([
        jnp.broadcast_to(mx2[0, 0] + mx2[0, HEADS] + mxe[0, HEADS], (16,)),
        jnp.broadcast_to(mx2[0, 1] + mx2[0, HEADS + 1] + mxe[0, HEADS + 1], (16,))])
    as2_flat = jnp.concatenate([asd2[:, 0], asd2[:, 1]])
    ad2_flat = jnp.concatenate([asd2[:, 2], asd2[:, 3]])
    num20, num21, den20, den21 = _sc_conv(
        2, xwh2.reshape(2 * N, OUT), src_p, dst_rows, aeP,
        as2_flat, ad2_flat, bounds2)

    out = pl.pallas_call(
        _k6_body,
        grid=(N_GRID,),
        in_specs=[pl.BlockSpec((NB, OUT), lambda i: (i, 0)),
                  pl.BlockSpec((NB, OUT), lambda i: (i, 0)),
                  pl.BlockSpec((NB, 1), lambda i: (i, 0)),
                  pl.BlockSpec((NB, 1), lambda i: (i, 0)),
                  _full_spec((HEADS * OUT,))],
        out_specs=pl.BlockSpec((NB, HEADS * OUT), lambda i: (i, 0)),
        out_shape=jax.ShapeDtypeStruct((N, HEADS * OUT), jnp.float32),
    )(num20, num21, den20.reshape(N, 1), den21.reshape(N, 1), bias2)
    return out
